# baseline (speedup 1.0000x reference)
import functools

import jax
import jax.numpy as jnp
from jax import lax
from jax.experimental import pallas as pl
from jax.experimental.pallas import tpu as pltpu
from jax.experimental.pallas import tpu_sc as plsc

f32 = jnp.float32
i32 = jnp.int32

N_NODES = 10000
NPAD = 10240
NFEAT = 256
FH = 128
HIDDEN = 512
NCLASS = 40
CPAD = 64
N_EDGES = 160000
E_ALL = N_EDGES + N_NODES
CHUNK = 128
CH_C = 88
CH_E = 44
EPAD = 32 * CH_E * CHUNK
DUMMY = N_NODES
RPT = NPAD // 16
MB = 512

_mesh = plsc.VectorSubcoreMesh(
    core_axis_name="c", subcore_axis_name="s", num_cores=2, num_subcores=16
)


def _deg_body(dst_hbm, degp_hbm, dst_v, deg_v):
    c = lax.axis_index("c")
    s = lax.axis_index("s")
    wid = c * 16 + s
    pltpu.sync_copy(dst_hbm.at[wid], dst_v)
    zeros16 = jnp.zeros((16,), f32)

    def zero_body(i, _):
        deg_v[pl.ds(i * 16, 16)] = zeros16
        return ()

    lax.fori_loop(0, NPAD // 16, zero_body, (), unroll=8)
    ones16 = jnp.ones((16,), f32)

    def cnt_body(i, _):
        idx = dst_v[pl.ds(i * 16, 16)]
        plsc.addupdate_scatter(deg_v, [idx], ones16)
        return ()

    lax.fori_loop(0, (CH_E * CHUNK) // 16, cnt_body, (), unroll=8)
    pltpu.sync_copy(deg_v, degp_hbm.at[wid])


_deg_kernel = pl.kernel(
    _deg_body,
    out_type=jax.ShapeDtypeStruct((32, NPAD), f32),
    mesh=_mesh,
    compiler_params=pltpu.CompilerParams(needs_layout_passes=False),
    scratch_types=[
        pltpu.VMEM((CH_E * CHUNK,), i32),
        pltpu.VMEM((NPAD,), f32),
    ],
)


def _prep_body(degp_ref, x_ref, xs0_ref, xs1_ref, dis_ref):
    deg = jnp.sum(degp_ref[...], axis=0)
    dis = lax.rsqrt(jnp.maximum(deg, 1.0))
    xs = x_ref[...] * dis[:, None]
    xs0_ref[...] = xs[:, :FH]
    xs1_ref[...] = xs[:, FH:]
    dis_ref[...] = jnp.broadcast_to(dis[:, None], dis_ref.shape)


_prep = pl.pallas_call(
    _prep_body,
    grid=(NPAD // MB,),
    in_specs=[
        pl.BlockSpec((32, MB), lambda m: (0, m)),
        pl.BlockSpec((MB, NFEAT), lambda m: (m, 0)),
    ],
    out_specs=[
        pl.BlockSpec((MB, FH), lambda m: (m, 0)),
        pl.BlockSpec((MB, FH), lambda m: (m, 0)),
        pl.BlockSpec((MB, FH), lambda m: (m, 0)),
    ],
    out_shape=[
        jax.ShapeDtypeStruct((NPAD, FH), f32),
        jax.ShapeDtypeStruct((NPAD, FH), f32),
        jax.ShapeDtypeStruct((NPAD, FH), f32),
    ],
)


def _pipelined_edge_loop(tab_hbm, src_v, dst_v, acc_sh, bufs, sems, n_chunks):
    nbuf = len(bufs)
    pend = [None] * nbuf
    for j in range(min(nbuf, n_chunks)):
        pend[j] = pltpu.async_copy(tab_hbm.at[src_v.at[j]], bufs[j], sems[j])
    for j in range(n_chunks):
        b = j % nbuf
        pend[b].wait()
        pltpu.sync_copy(bufs[b], acc_sh.at[dst_v.at[j]], add=True)
        jn = j + nbuf
        if jn < n_chunks:
            pend[b] = pltpu.async_copy(tab_hbm.at[src_v.at[jn]], bufs[b], sems[b])


def _drain_acc(acc_sh, out_hbm, base, out_base, bufs, sems):
    n = RPT // CHUNK
    pltpu.async_copy(acc_sh.at[pl.ds(base, CHUNK)], bufs[0], sems[0])
    for k in range(n):
        b = k % 2
        pltpu.make_async_copy(
            acc_sh.at[pl.ds(base + k * CHUNK, CHUNK)], bufs[b], sems[b]
        ).wait()
        if k + 1 < n:
            pltpu.async_copy(
                acc_sh.at[pl.ds(base + (k + 1) * CHUNK, CHUNK)],
                bufs[(k + 1) % 2], sems[(k + 1) % 2],
            )
        pltpu.sync_copy(bufs[b], out_hbm.at[pl.ds(out_base + k * CHUNK, CHUNK)])


GC = 8


def _agg1_body(xs0_hbm, xs1_hbm, src_hbm, dst_hbm, zero_hbm, out_hbm,
               si0, si1, di0, di1, r0, r1, acc_sh, gs0, gs1, s0, s1):
    bufs = (r0, r1)
    sems = (s0, s1)
    sibufs = (si0, si1)
    dibufs = (di0, di1)
    gsems = (gs0, gs1)
    c = lax.axis_index("c")
    s = lax.axis_index("s")
    pltpu.sync_copy(zero_hbm, r0)
    base = s * RPT
    for k in range(RPT // CHUNK):
        pltpu.sync_copy(r0, acc_sh.at[pl.ds(base + k * CHUNK, CHUNK)])
    plsc.subcore_barrier()

    def edge_loop(tab):
        ng = CH_C // GC
        pend_i = [None, None]

        def start_idx(g):
            gb = g % 2
            a = pltpu.async_copy(src_hbm.at[s, g], sibufs[gb], gsems[gb])
            b = pltpu.async_copy(dst_hbm.at[s, g], dibufs[gb], gsems[gb])
            pend_i[gb] = (a, b)

        start_idx(0)
        rpend = [None, None]
        for g in range(ng):
            gb = g % 2
            ia, ib = pend_i[gb]
            ia.wait()
            ib.wait()
            if g + 1 < ng:
                start_idx(g + 1)
            si, di = sibufs[gb], dibufs[gb]
            rpend[0] = pltpu.async_copy(tab.at[si.at[0]], bufs[0], sems[0])
            for k in range(GC):
                rb = k % 2
                if k + 1 < GC:
                    rpend[1 - rb] = pltpu.async_copy(
                        tab.at[si.at[k + 1]], bufs[1 - rb], sems[1 - rb])
                rpend[rb].wait()
                pltpu.sync_copy(bufs[rb], acc_sh.at[di.at[k]], add=True)

    @pl.when(c == 0)
    def _():
        edge_loop(xs0_hbm)

    @pl.when(c == 1)
    def _():
        edge_loop(xs1_hbm)

    plsc.subcore_barrier()
    _drain_acc(acc_sh, out_hbm, base, c * NPAD + base, bufs, sems)


_agg1_kernel = pl.kernel(
    _agg1_body,
    out_type=jax.ShapeDtypeStruct((2 * NPAD, FH), f32),
    mesh=_mesh,
    scratch_types=[
        pltpu.VMEM((GC, CHUNK), i32),
        pltpu.VMEM((GC, CHUNK), i32),
        pltpu.VMEM((GC, CHUNK), i32),
        pltpu.VMEM((GC, CHUNK), i32),
        pltpu.VMEM((CHUNK, FH), f32),
        pltpu.VMEM((CHUNK, FH), f32),
        pltpu.VMEM_SHARED((NPAD, FH), f32),
        pltpu.SemaphoreType.DMA,
        pltpu.SemaphoreType.DMA,
        pltpu.SemaphoreType.DMA,
        pltpu.SemaphoreType.DMA,
    ],
)


def _dense_body(a0_ref, a1_ref, dis_ref, w1_ref, b1_ref, w2_ref, ys_ref):
    d = dis_ref[...]
    a0 = a0_ref[...] * d
    a1 = a1_ref[...] * d
    w1 = w1_ref[...]
    h = (
        jnp.dot(a0, w1[:FH], preferred_element_type=f32,
                precision=lax.Precision.HIGHEST)
        + jnp.dot(a1, w1[FH:], preferred_element_type=f32,
                  precision=lax.Precision.HIGHEST)
    )
    h = jnp.maximum(h + b1_ref[...], 0.0)
    y = jnp.dot(h, w2_ref[...], preferred_element_type=f32,
                precision=lax.Precision.HIGHEST)
    ys_ref[...] = y * d[:, :CPAD]


_dense = pl.pallas_call(
    _dense_body,
    grid=(NPAD // MB,),
    in_specs=[
        pl.BlockSpec((MB, FH), lambda m: (m, 0)),
        pl.BlockSpec((MB, FH), lambda m: (m + NPAD // MB, 0)),
        pl.BlockSpec((MB, FH), lambda m: (m, 0)),
        pl.BlockSpec((NFEAT, HIDDEN), lambda m: (0, 0)),
        pl.BlockSpec((1, HIDDEN), lambda m: (0, 0)),
        pl.BlockSpec((HIDDEN, CPAD), lambda m: (0, 0)),
    ],
    out_specs=pl.BlockSpec((MB, CPAD), lambda m: (m, 0)),
    out_shape=jax.ShapeDtypeStruct((NPAD, CPAD), f32),
)


def _agg2_body(ys_hbm, src_hbm, dst_hbm, zero_hbm, out_hbm,
               src_v, dst_v, r0, r1, r2, acc_sh, s0, s1, s2):
    bufs = (r0, r1, r2)
    sems = (s0, s1, s2)
    c = lax.axis_index("c")
    s = lax.axis_index("s")
    wid = c * 16 + s
    pltpu.sync_copy(src_hbm.at[wid], src_v)
    pltpu.sync_copy(dst_hbm.at[wid], dst_v)
    pltpu.sync_copy(zero_hbm, r0)
    base = s * RPT
    for k in range(RPT // CHUNK):
        pltpu.sync_copy(r0, acc_sh.at[pl.ds(base + k * CHUNK, CHUNK)])
    plsc.subcore_barrier()
    _pipelined_edge_loop(ys_hbm, src_v, dst_v, acc_sh, bufs, sems, CH_E)
    plsc.subcore_barrier()
    _drain_acc(acc_sh, out_hbm, base, c * NPAD + base, bufs, sems)


_agg2_kernel = pl.kernel(
    _agg2_body,
    out_type=jax.ShapeDtypeStruct((2 * NPAD, CPAD), f32),
    mesh=_mesh,
    compiler_params=pltpu.CompilerParams(use_tc_tiling_on_sc=False),
    scratch_types=[
        pltpu.VMEM((CH_E, CHUNK), i32),
        pltpu.VMEM((CH_E, CHUNK), i32),
        pltpu.VMEM((CHUNK, CPAD), f32),
        pltpu.VMEM((CHUNK, CPAD), f32),
        pltpu.VMEM((CHUNK, CPAD), f32),
        pltpu.VMEM_SHARED((NPAD, CPAD), f32),
        pltpu.SemaphoreType.DMA,
        pltpu.SemaphoreType.DMA,
        pltpu.SemaphoreType.DMA,
    ],
)


def _out_body(a0_ref, a1_ref, dis_ref, b2_ref, o_ref):
    z = (a0_ref[...] + a1_ref[...]) * dis_ref[...][:, :CPAD] + b2_ref[...]
    col = lax.broadcasted_iota(i32, z.shape, 1)
    z = jnp.where(col < NCLASS, z, -1e30)
    m = jnp.max(z, axis=1, keepdims=True)
    e = jnp.exp(z - m)
    ssum = jnp.sum(e, axis=1, keepdims=True)
    o_ref[...] = z - m - jnp.log(ssum)


_outk = pl.pallas_call(
    _out_body,
    grid=(NPAD // MB,),
    in_specs=[
        pl.BlockSpec((MB, CPAD), lambda m: (m, 0)),
        pl.BlockSpec((MB, CPAD), lambda m: (m + NPAD // MB, 0)),
        pl.BlockSpec((MB, FH), lambda m: (m, 0)),
        pl.BlockSpec((1, CPAD), lambda m: (0, 0)),
    ],
    out_specs=pl.BlockSpec((MB, CPAD), lambda m: (m, 0)),
    out_shape=jax.ShapeDtypeStruct((NPAD, CPAD), f32),
)


def kernel(x, edge_index, W1, b1, W2, b2):
    src = edge_index[0].astype(i32)
    dst = edge_index[1].astype(i32)
    loop = jnp.arange(N_NODES, dtype=i32)
    n_pad_edges = EPAD - E_ALL
    src_pad = jnp.concatenate([src, loop, jnp.zeros((n_pad_edges,), i32)])
    dst_pad = jnp.concatenate([dst, loop, jnp.full((n_pad_edges,), DUMMY, i32)])
    srcC = src_pad.reshape(16, CH_C // GC, GC, CHUNK)
    dstC = dst_pad.reshape(16, CH_C // GC, GC, CHUNK)
    srcE = src_pad.reshape(32, CH_E, CHUNK)
    dstE = dst_pad.reshape(32, CH_E, CHUNK)
    dstA = dst_pad.reshape(32, CH_E * CHUNK)
    x_pad = jnp.concatenate([x, jnp.zeros((NPAD - N_NODES, NFEAT), f32)])
    w2p = jnp.pad(W2, ((0, 0), (0, CPAD - NCLASS)))
    b2p = jnp.pad(b2, (0, CPAD - NCLASS)).reshape(1, CPAD)
    b1r = b1.reshape(1, HIDDEN)
    zero128 = jnp.zeros((CHUNK, FH), f32)
    zero64 = jnp.zeros((CHUNK, CPAD), f32)

    degp = _deg_kernel(dstA)
    xs0, xs1, dis2d = _prep(degp, x_pad)
    agg = _agg1_kernel(xs0, xs1, srcC, dstC, zero128)
    ys = _dense(agg, agg, dis2d, W1, b1r, w2p)
    acc2 = _agg2_kernel(ys, srcE, dstE, zero64)
    o = _outk(acc2, acc2, dis2d, b2p)
    return o[:N_NODES, :NCLASS]


# speedup vs baseline: 1.0043x; 1.0043x over previous
import functools

import jax
import jax.numpy as jnp
from jax import lax
from jax.experimental import pallas as pl
from jax.experimental.pallas import tpu as pltpu
from jax.experimental.pallas import tpu_sc as plsc

f32 = jnp.float32
i32 = jnp.int32

N_NODES = 10000
NPAD = 10240
NFEAT = 256
FH = 128
HIDDEN = 512
NCLASS = 40
CPAD = 64
N_EDGES = 160000
E_ALL = N_EDGES + N_NODES
CHUNK = 128
CH_C = 88
CH_E = 44
EPAD = 32 * CH_E * CHUNK
DUMMY = N_NODES
RPT = NPAD // 16
MB = 512

_mesh = plsc.VectorSubcoreMesh(
    core_axis_name="c", subcore_axis_name="s", num_cores=2, num_subcores=16
)


def _deg_body(dst_hbm, degp_hbm, dst_v, deg_v):
    c = lax.axis_index("c")
    s = lax.axis_index("s")
    wid = c * 16 + s
    pltpu.sync_copy(dst_hbm.at[wid], dst_v)
    zeros16 = jnp.zeros((16,), f32)

    def zero_body(i, _):
        deg_v[pl.ds(i * 16, 16)] = zeros16
        return ()

    lax.fori_loop(0, NPAD // 16, zero_body, (), unroll=8)
    ones16 = jnp.ones((16,), f32)

    def cnt_body(i, _):
        idx = dst_v[pl.ds(i * 16, 16)]
        plsc.addupdate_scatter(deg_v, [idx], ones16)
        return ()

    lax.fori_loop(0, (CH_E * CHUNK) // 16, cnt_body, (), unroll=8)
    pltpu.sync_copy(deg_v, degp_hbm.at[wid])


_deg_kernel = pl.kernel(
    _deg_body,
    out_type=jax.ShapeDtypeStruct((32, NPAD), f32),
    mesh=_mesh,
    compiler_params=pltpu.CompilerParams(needs_layout_passes=False),
    scratch_types=[
        pltpu.VMEM((CH_E * CHUNK,), i32),
        pltpu.VMEM((NPAD,), f32),
    ],
)


def _prep_body(degp_ref, x_ref, xs0_ref, xs1_ref, dis_ref):
    deg = jnp.sum(degp_ref[...], axis=0)
    dis = lax.rsqrt(jnp.maximum(deg, 1.0))
    xs = x_ref[...] * dis[:, None]
    xs0_ref[...] = xs[:, :FH]
    xs1_ref[...] = xs[:, FH:]
    dis_ref[...] = jnp.broadcast_to(dis[:, None], dis_ref.shape)


_prep = pl.pallas_call(
    _prep_body,
    grid=(NPAD // MB,),
    in_specs=[
        pl.BlockSpec((32, MB), lambda m: (0, m)),
        pl.BlockSpec((MB, NFEAT), lambda m: (m, 0)),
    ],
    out_specs=[
        pl.BlockSpec((MB, FH), lambda m: (m, 0)),
        pl.BlockSpec((MB, FH), lambda m: (m, 0)),
        pl.BlockSpec((MB, FH), lambda m: (m, 0)),
    ],
    out_shape=[
        jax.ShapeDtypeStruct((NPAD, FH), f32),
        jax.ShapeDtypeStruct((NPAD, FH), f32),
        jax.ShapeDtypeStruct((NPAD, FH), f32),
    ],
)


def _pipelined_edge_loop(tab_hbm, src_v, dst_v, acc_sh, bufs, sems, n_chunks):
    r0, r1 = bufs[0], bufs[1]
    s0, s1 = sems[0], sems[1]
    pltpu.async_copy(tab_hbm.at[src_v.at[0]], r0, s0)

    def body(j, _):
        def step(rb, sb, ro, so):
            @pl.when(j + 1 < n_chunks)
            def _():
                pltpu.async_copy(tab_hbm.at[src_v.at[j + 1]], ro, so)

            pltpu.make_async_copy(tab_hbm.at[src_v.at[j]], rb, sb).wait()
            pltpu.sync_copy(rb, acc_sh.at[dst_v.at[j]], add=True)

        @pl.when(j % 2 == 0)
        def _():
            step(r0, s0, r1, s1)

        @pl.when(j % 2 == 1)
        def _():
            step(r1, s1, r0, s0)

        return ()

    lax.fori_loop(0, n_chunks, body, ())


def _drain_acc(acc_sh, out_hbm, base, out_base, bufs, sems):
    n = RPT // CHUNK
    pltpu.async_copy(acc_sh.at[pl.ds(base, CHUNK)], bufs[0], sems[0])
    for k in range(n):
        b = k % 2
        pltpu.make_async_copy(
            acc_sh.at[pl.ds(base + k * CHUNK, CHUNK)], bufs[b], sems[b]
        ).wait()
        if k + 1 < n:
            pltpu.async_copy(
                acc_sh.at[pl.ds(base + (k + 1) * CHUNK, CHUNK)],
                bufs[(k + 1) % 2], sems[(k + 1) % 2],
            )
        pltpu.sync_copy(bufs[b], out_hbm.at[pl.ds(out_base + k * CHUNK, CHUNK)])


GC = 8


def _agg1_body(xs0_hbm, xs1_hbm, src_hbm, dst_hbm, zero_hbm, out_hbm,
               si0, si1, di0, di1, r0, r1, acc_sh, gs0, gs1, s0, s1):
    bufs = (r0, r1)
    sems = (s0, s1)
    sibufs = (si0, si1)
    dibufs = (di0, di1)
    gsems = (gs0, gs1)
    c = lax.axis_index("c")
    s = lax.axis_index("s")
    pltpu.sync_copy(zero_hbm, r0)
    base = s * RPT
    for k in range(RPT // CHUNK):
        pltpu.sync_copy(r0, acc_sh.at[pl.ds(base + k * CHUNK, CHUNK)])
    plsc.subcore_barrier()

    def edge_loop(tab):
        ng = CH_C // GC
        pltpu.async_copy(src_hbm.at[s, 0], si0, gs0)
        pltpu.async_copy(dst_hbm.at[s, 0], di0, gs0)

        def gbody(g, _):
            def run_group(si, di, gsem, sin, din, gsemn):
                pltpu.make_async_copy(src_hbm.at[s, g], si, gsem).wait()
                pltpu.make_async_copy(dst_hbm.at[s, g], di, gsem).wait()

                @pl.when(g + 1 < ng)
                def _():
                    pltpu.async_copy(src_hbm.at[s, g + 1], sin, gsemn)
                    pltpu.async_copy(dst_hbm.at[s, g + 1], din, gsemn)

                pend = [None, None]
                pend[0] = pltpu.async_copy(tab.at[si.at[0]], bufs[0], sems[0])
                for k in range(GC):
                    rb = k % 2
                    if k + 1 < GC:
                        pend[1 - rb] = pltpu.async_copy(
                            tab.at[si.at[k + 1]], bufs[1 - rb], sems[1 - rb])
                    pend[rb].wait()
                    pltpu.sync_copy(bufs[rb], acc_sh.at[di.at[k]], add=True)

            @pl.when(g % 2 == 0)
            def _():
                run_group(si0, di0, gs0, si1, di1, gs1)

            @pl.when(g % 2 == 1)
            def _():
                run_group(si1, di1, gs1, si0, di0, gs0)

            return ()

        lax.fori_loop(0, ng, gbody, ())

    @pl.when(c == 0)
    def _():
        edge_loop(xs0_hbm)

    @pl.when(c == 1)
    def _():
        edge_loop(xs1_hbm)

    plsc.subcore_barrier()
    _drain_acc(acc_sh, out_hbm, base, c * NPAD + base, bufs, sems)


_agg1_kernel = pl.kernel(
    _agg1_body,
    out_type=jax.ShapeDtypeStruct((2 * NPAD, FH), f32),
    mesh=_mesh,
    scratch_types=[
        pltpu.VMEM((GC, CHUNK), i32),
        pltpu.VMEM((GC, CHUNK), i32),
        pltpu.VMEM((GC, CHUNK), i32),
        pltpu.VMEM((GC, CHUNK), i32),
        pltpu.VMEM((CHUNK, FH), f32),
        pltpu.VMEM((CHUNK, FH), f32),
        pltpu.VMEM_SHARED((NPAD, FH), f32),
        pltpu.SemaphoreType.DMA,
        pltpu.SemaphoreType.DMA,
        pltpu.SemaphoreType.DMA,
        pltpu.SemaphoreType.DMA,
    ],
)


def _dense_body(a0_ref, a1_ref, dis_ref, w1_ref, b1_ref, w2_ref, ys_ref):
    d = dis_ref[...]
    a0 = a0_ref[...] * d
    a1 = a1_ref[...] * d
    w1 = w1_ref[...]
    h = (
        jnp.dot(a0, w1[:FH], preferred_element_type=f32,
                precision=lax.Precision.HIGHEST)
        + jnp.dot(a1, w1[FH:], preferred_element_type=f32,
                  precision=lax.Precision.HIGHEST)
    )
    h = jnp.maximum(h + b1_ref[...], 0.0)
    y = jnp.dot(h, w2_ref[...], preferred_element_type=f32,
                precision=lax.Precision.HIGHEST)
    ys_ref[...] = y * d[:, :CPAD]


_dense = pl.pallas_call(
    _dense_body,
    grid=(NPAD // MB,),
    in_specs=[
        pl.BlockSpec((MB, FH), lambda m: (m, 0)),
        pl.BlockSpec((MB, FH), lambda m: (m + NPAD // MB, 0)),
        pl.BlockSpec((MB, FH), lambda m: (m, 0)),
        pl.BlockSpec((NFEAT, HIDDEN), lambda m: (0, 0)),
        pl.BlockSpec((1, HIDDEN), lambda m: (0, 0)),
        pl.BlockSpec((HIDDEN, CPAD), lambda m: (0, 0)),
    ],
    out_specs=pl.BlockSpec((MB, CPAD), lambda m: (m, 0)),
    out_shape=jax.ShapeDtypeStruct((NPAD, CPAD), f32),
)


def _agg2_body(ys_hbm, src_hbm, dst_hbm, zero_hbm, out_hbm,
               src_v, dst_v, r0, r1, acc_sh, s0, s1):
    bufs = (r0, r1)
    sems = (s0, s1)
    c = lax.axis_index("c")
    s = lax.axis_index("s")
    wid = c * 16 + s
    pltpu.sync_copy(src_hbm.at[wid], src_v)
    pltpu.sync_copy(dst_hbm.at[wid], dst_v)
    pltpu.sync_copy(zero_hbm, r0)
    base = s * RPT
    for k in range(RPT // CHUNK):
        pltpu.sync_copy(r0, acc_sh.at[pl.ds(base + k * CHUNK, CHUNK)])
    plsc.subcore_barrier()
    _pipelined_edge_loop(ys_hbm, src_v, dst_v, acc_sh, bufs, sems, CH_E)
    plsc.subcore_barrier()
    _drain_acc(acc_sh, out_hbm, base, c * NPAD + base, bufs, sems)


_agg2_kernel = pl.kernel(
    _agg2_body,
    out_type=jax.ShapeDtypeStruct((2 * NPAD, CPAD), f32),
    mesh=_mesh,
    compiler_params=pltpu.CompilerParams(use_tc_tiling_on_sc=False),
    scratch_types=[
        pltpu.VMEM((CH_E, CHUNK), i32),
        pltpu.VMEM((CH_E, CHUNK), i32),
        pltpu.VMEM((CHUNK, CPAD), f32),
        pltpu.VMEM((CHUNK, CPAD), f32),
        pltpu.VMEM_SHARED((NPAD, CPAD), f32),
        pltpu.SemaphoreType.DMA,
        pltpu.SemaphoreType.DMA,
    ],
)


def _out_body(a0_ref, a1_ref, dis_ref, b2_ref, o_ref):
    z = (a0_ref[...] + a1_ref[...]) * dis_ref[...][:, :CPAD] + b2_ref[...]
    col = lax.broadcasted_iota(i32, z.shape, 1)
    z = jnp.where(col < NCLASS, z, -1e30)
    m = jnp.max(z, axis=1, keepdims=True)
    e = jnp.exp(z - m)
    ssum = jnp.sum(e, axis=1, keepdims=True)
    o_ref[...] = z - m - jnp.log(ssum)


_outk = pl.pallas_call(
    _out_body,
    grid=(NPAD // MB,),
    in_specs=[
        pl.BlockSpec((MB, CPAD), lambda m: (m, 0)),
        pl.BlockSpec((MB, CPAD), lambda m: (m + NPAD // MB, 0)),
        pl.BlockSpec((MB, FH), lambda m: (m, 0)),
        pl.BlockSpec((1, CPAD), lambda m: (0, 0)),
    ],
    out_specs=pl.BlockSpec((MB, CPAD), lambda m: (m, 0)),
    out_shape=jax.ShapeDtypeStruct((NPAD, CPAD), f32),
)


def kernel(x, edge_index, W1, b1, W2, b2):
    src = edge_index[0].astype(i32)
    dst = edge_index[1].astype(i32)
    loop = jnp.arange(N_NODES, dtype=i32)
    n_pad_edges = EPAD - E_ALL
    src_pad = jnp.concatenate([src, loop, jnp.zeros((n_pad_edges,), i32)])
    dst_pad = jnp.concatenate([dst, loop, jnp.full((n_pad_edges,), DUMMY, i32)])
    srcC = src_pad.reshape(16, CH_C // GC, GC, CHUNK)
    dstC = dst_pad.reshape(16, CH_C // GC, GC, CHUNK)
    srcE = src_pad.reshape(32, CH_E, CHUNK)
    dstE = dst_pad.reshape(32, CH_E, CHUNK)
    dstA = dst_pad.reshape(32, CH_E * CHUNK)
    x_pad = jnp.concatenate([x, jnp.zeros((NPAD - N_NODES, NFEAT), f32)])
    w2p = jnp.pad(W2, ((0, 0), (0, CPAD - NCLASS)))
    b2p = jnp.pad(b2, (0, CPAD - NCLASS)).reshape(1, CPAD)
    b1r = b1.reshape(1, HIDDEN)
    zero128 = jnp.zeros((CHUNK, FH), f32)
    zero64 = jnp.zeros((CHUNK, CPAD), f32)

    degp = _deg_kernel(dstA)
    xs0, xs1, dis2d = _prep(degp, x_pad)
    agg = _agg1_kernel(xs0, xs1, srcC, dstC, zero128)
    ys = _dense(agg, agg, dis2d, W1, b1r, w2p)
    acc2 = _agg2_kernel(ys, srcE, dstE, zero64)
    o = _outk(acc2, acc2, dis2d, b2p)
    return o[:N_NODES, :NCLASS]


# speedup vs baseline: 1.4889x; 1.4826x over previous
import jax
import jax.numpy as jnp
from jax import lax
from jax.experimental import pallas as pl
from jax.experimental.pallas import tpu as pltpu
from jax.experimental.pallas import tpu_sc as plsc

f32 = jnp.float32
i32 = jnp.int32

N_NODES = 10000
NPAD = 10240
NFEAT = 256
FH = 128
HIDDEN = 512
NCLASS = 40
CPAD = 64
N_EDGES = 160000
CHUNK = 128
CH_C = 80
CH_E = 40
EPAD = 32 * CH_E * CHUNK
DUMMY = N_NODES
RPT = NPAD // 16
MB = 512

_mesh = plsc.VectorSubcoreMesh(
    core_axis_name="c", subcore_axis_name="s", num_cores=2, num_subcores=16
)


def _deg_body(dst_hbm, degp_hbm, dst_v, deg_v):
    c = lax.axis_index("c")
    s = lax.axis_index("s")
    wid = c * 16 + s
    pltpu.sync_copy(dst_hbm.at[wid], dst_v)
    zeros16 = jnp.zeros((16,), f32)

    def zero_body(i, _):
        deg_v[pl.ds(i * 16, 16)] = zeros16
        return ()

    lax.fori_loop(0, NPAD // 16, zero_body, (), unroll=8)
    ones16 = jnp.ones((16,), f32)

    def cnt_body(i, _):
        idx = dst_v[pl.ds(i * 16, 16)]
        plsc.addupdate_scatter(deg_v, [idx], ones16)
        return ()

    lax.fori_loop(0, (CH_E * CHUNK) // 16, cnt_body, (), unroll=8)
    pltpu.sync_copy(deg_v, degp_hbm.at[wid])


_deg_kernel = pl.kernel(
    _deg_body,
    out_type=jax.ShapeDtypeStruct((32, NPAD), f32),
    mesh=_mesh,
    compiler_params=pltpu.CompilerParams(needs_layout_passes=False),
    scratch_types=[
        pltpu.VMEM((CH_E * CHUNK,), i32),
        pltpu.VMEM((NPAD,), f32),
    ],
)


def _prep_body(degp_ref, x_ref, xs0_ref, xs1_ref, dis_ref):
    deg = jnp.sum(degp_ref[...], axis=0) + 1.0
    dis = lax.rsqrt(deg)
    xs = x_ref[...] * dis[:, None]
    xs0_ref[...] = xs[:, :FH]
    xs1_ref[...] = xs[:, FH:]
    dis_ref[...] = jnp.broadcast_to(dis[:, None], dis_ref.shape)


_prep = pl.pallas_call(
    _prep_body,
    grid=(NPAD // MB,),
    in_specs=[
        pl.BlockSpec((32, MB), lambda m: (0, m)),
        pl.BlockSpec((MB, NFEAT), lambda m: (m, 0)),
    ],
    out_specs=[
        pl.BlockSpec((MB, FH), lambda m: (m, 0)),
        pl.BlockSpec((MB, FH), lambda m: (m, 0)),
        pl.BlockSpec((MB, FH), lambda m: (m, 0)),
    ],
    out_shape=[
        jax.ShapeDtypeStruct((NPAD, FH), f32),
        jax.ShapeDtypeStruct((NPAD, FH), f32),
        jax.ShapeDtypeStruct((NPAD, FH), f32),
    ],
)


def _serial_edge_loop(tab_hbm, src_v, dst_v, acc_sh, rows_v, sem, n_chunks):
    def body(j, _):
        pltpu.async_copy(tab_hbm.at[src_v.at[j]], rows_v, sem).wait()
        pltpu.sync_copy(rows_v, acc_sh.at[dst_v.at[j]], add=True)
        return ()

    lax.fori_loop(0, n_chunks, body, ())


def _drain_acc(acc_sh, out_hbm, base, out_base, rows_v):
    for k in range(RPT // CHUNK):
        pltpu.sync_copy(acc_sh.at[pl.ds(base + k * CHUNK, CHUNK)], rows_v)
        pltpu.sync_copy(rows_v, out_hbm.at[pl.ds(out_base + k * CHUNK, CHUNK)])


def _agg1_body(xs0_hbm, xs1_hbm, src_hbm, dst_hbm, out_hbm,
               src_v, dst_v, rows_v, acc_sh, sem):
    c = lax.axis_index("c")
    s = lax.axis_index("s")
    pltpu.sync_copy(src_hbm.at[s], src_v)
    pltpu.sync_copy(dst_hbm.at[s], dst_v)
    base = s * RPT

    def init_and_run(tab_hbm):
        for k in range(RPT // CHUNK):
            pltpu.sync_copy(tab_hbm.at[pl.ds(base + k * CHUNK, CHUNK)], rows_v)
            pltpu.sync_copy(rows_v, acc_sh.at[pl.ds(base + k * CHUNK, CHUNK)])
        plsc.subcore_barrier()
        _serial_edge_loop(tab_hbm, src_v, dst_v, acc_sh, rows_v, sem, CH_C)

    @pl.when(c == 0)
    def _():
        init_and_run(xs0_hbm)

    @pl.when(c == 1)
    def _():
        init_and_run(xs1_hbm)

    plsc.subcore_barrier()
    _drain_acc(acc_sh, out_hbm, base, c * NPAD + base, rows_v)


_agg1_kernel = pl.kernel(
    _agg1_body,
    out_type=jax.ShapeDtypeStruct((2 * NPAD, FH), f32),
    mesh=_mesh,
    scratch_types=[
        pltpu.VMEM((CH_C, CHUNK), i32),
        pltpu.VMEM((CH_C, CHUNK), i32),
        pltpu.VMEM((CHUNK, FH), f32),
        pltpu.VMEM_SHARED((NPAD, FH), f32),
        pltpu.SemaphoreType.DMA,
    ],
)


def _dense_body(a0_ref, a1_ref, dis_ref, w1_ref, b1_ref, w2_ref, ys_ref):
    d = dis_ref[...]
    a0 = a0_ref[...] * d
    a1 = a1_ref[...] * d
    w1 = w1_ref[...]
    h = (
        jnp.dot(a0, w1[:FH], preferred_element_type=f32,
                precision=lax.Precision.HIGHEST)
        + jnp.dot(a1, w1[FH:], preferred_element_type=f32,
                  precision=lax.Precision.HIGHEST)
    )
    h = jnp.maximum(h + b1_ref[...], 0.0)
    y = jnp.dot(h, w2_ref[...], preferred_element_type=f32,
                precision=lax.Precision.HIGHEST)
    ys_ref[...] = y * d[:, :CPAD]


_dense = pl.pallas_call(
    _dense_body,
    grid=(NPAD // MB,),
    in_specs=[
        pl.BlockSpec((MB, FH), lambda m: (m, 0)),
        pl.BlockSpec((MB, FH), lambda m: (m + NPAD // MB, 0)),
        pl.BlockSpec((MB, FH), lambda m: (m, 0)),
        pl.BlockSpec((NFEAT, HIDDEN), lambda m: (0, 0)),
        pl.BlockSpec((1, HIDDEN), lambda m: (0, 0)),
        pl.BlockSpec((HIDDEN, CPAD), lambda m: (0, 0)),
    ],
    out_specs=pl.BlockSpec((MB, CPAD), lambda m: (m, 0)),
    out_shape=jax.ShapeDtypeStruct((NPAD, CPAD), f32),
)


def _agg2_body(ys_hbm, src_hbm, dst_hbm, zero_hbm, out_hbm,
               src_v, dst_v, rows_v, acc_sh, sem):
    c = lax.axis_index("c")
    s = lax.axis_index("s")
    wid = c * 16 + s
    pltpu.sync_copy(src_hbm.at[wid], src_v)
    pltpu.sync_copy(dst_hbm.at[wid], dst_v)
    pltpu.sync_copy(zero_hbm, rows_v)
    base = s * RPT
    for k in range(RPT // CHUNK):
        pltpu.sync_copy(rows_v, acc_sh.at[pl.ds(base + k * CHUNK, CHUNK)])
    plsc.subcore_barrier()
    _serial_edge_loop(ys_hbm, src_v, dst_v, acc_sh, rows_v, sem, CH_E)
    plsc.subcore_barrier()
    _drain_acc(acc_sh, out_hbm, base, c * NPAD + base, rows_v)


_agg2_kernel = pl.kernel(
    _agg2_body,
    out_type=jax.ShapeDtypeStruct((2 * NPAD, CPAD), f32),
    mesh=_mesh,
    compiler_params=pltpu.CompilerParams(use_tc_tiling_on_sc=False),
    scratch_types=[
        pltpu.VMEM((CH_E, CHUNK), i32),
        pltpu.VMEM((CH_E, CHUNK), i32),
        pltpu.VMEM((CHUNK, CPAD), f32),
        pltpu.VMEM_SHARED((NPAD, CPAD), f32),
        pltpu.SemaphoreType.DMA,
    ],
)


def _out_body(a0_ref, a1_ref, ys_ref, dis_ref, b2_ref, o_ref):
    z = (a0_ref[...] + a1_ref[...] + ys_ref[...]) * dis_ref[...][:, :CPAD]
    z = z + b2_ref[...]
    col = lax.broadcasted_iota(i32, z.shape, 1)
    z = jnp.where(col < NCLASS, z, -1e30)
    m = jnp.max(z, axis=1, keepdims=True)
    e = jnp.exp(z - m)
    ssum = jnp.sum(e, axis=1, keepdims=True)
    o_ref[...] = z - m - jnp.log(ssum)


_outk = pl.pallas_call(
    _out_body,
    grid=(NPAD // MB,),
    in_specs=[
        pl.BlockSpec((MB, CPAD), lambda m: (m, 0)),
        pl.BlockSpec((MB, CPAD), lambda m: (m + NPAD // MB, 0)),
        pl.BlockSpec((MB, CPAD), lambda m: (m, 0)),
        pl.BlockSpec((MB, FH), lambda m: (m, 0)),
        pl.BlockSpec((1, CPAD), lambda m: (0, 0)),
    ],
    out_specs=pl.BlockSpec((MB, CPAD), lambda m: (m, 0)),
    out_shape=jax.ShapeDtypeStruct((NPAD, CPAD), f32),
)


def kernel(x, edge_index, W1, b1, W2, b2):
    src = edge_index[0].astype(i32)
    dst = edge_index[1].astype(i32)
    n_pad_edges = EPAD - N_EDGES
    src_pad = jnp.concatenate([src, jnp.zeros((n_pad_edges,), i32)])
    dst_pad = jnp.concatenate([dst, jnp.full((n_pad_edges,), DUMMY, i32)])
    srcC = src_pad.reshape(16, CH_C, CHUNK)
    dstC = dst_pad.reshape(16, CH_C, CHUNK)
    srcE = src_pad.reshape(32, CH_E, CHUNK)
    dstE = dst_pad.reshape(32, CH_E, CHUNK)
    dstA = dst_pad.reshape(32, CH_E * CHUNK)
    x_pad = jnp.concatenate([x, jnp.zeros((NPAD - N_NODES, NFEAT), f32)])
    w2p = jnp.pad(W2, ((0, 0), (0, CPAD - NCLASS)))
    b2p = jnp.pad(b2, (0, CPAD - NCLASS)).reshape(1, CPAD)
    b1r = b1.reshape(1, HIDDEN)
    zero64 = jnp.zeros((CHUNK, CPAD), f32)

    degp = _deg_kernel(dstA)
    xs0, xs1, dis2d = _prep(degp, x_pad)
    agg = _agg1_kernel(xs0, xs1, srcC, dstC)
    ys = _dense(agg, agg, dis2d, W1, b1r, w2p)
    acc2 = _agg2_kernel(ys, srcE, dstE, zero64)
    o = _outk(acc2, acc2, ys, dis2d, b2p)
    return o[:N_NODES, :NCLASS]


# speedup vs baseline: 2.5712x; 1.7268x over previous
import jax
import jax.numpy as jnp
from jax import lax
from jax.experimental import pallas as pl
from jax.experimental.pallas import tpu as pltpu
from jax.experimental.pallas import tpu_sc as plsc

f32 = jnp.float32
i32 = jnp.int32

N_NODES = 10000
NPAD = 10240
NFEAT = 256
FH = 128
HIDDEN = 512
NCLASS = 40
CPAD = 64
N_EDGES = 160000
CHUNK = 128
CH_C = 80
CH_E = 40
EPAD = 32 * CH_E * CHUNK
DUMMY = N_NODES
RPT = NPAD // 16
MB = 512

_mesh = plsc.VectorSubcoreMesh(
    core_axis_name="c", subcore_axis_name="s", num_cores=2, num_subcores=16
)


def _deg_body(dst_hbm, degp_hbm, dst_v, deg_v):
    c = lax.axis_index("c")
    s = lax.axis_index("s")
    wid = c * 16 + s
    pltpu.sync_copy(dst_hbm.at[wid], dst_v)
    zeros16 = jnp.zeros((16,), f32)

    def zero_body(i, _):
        deg_v[pl.ds(i * 16, 16)] = zeros16
        return ()

    lax.fori_loop(0, NPAD // 16, zero_body, (), unroll=8)
    ones16 = jnp.ones((16,), f32)

    def cnt_body(i, _):
        idx = dst_v[pl.ds(i * 16, 16)]
        plsc.addupdate_scatter(deg_v, [idx], ones16)
        return ()

    lax.fori_loop(0, (CH_E * CHUNK) // 16, cnt_body, (), unroll=8)
    pltpu.sync_copy(deg_v, degp_hbm.at[wid])


_deg_kernel = pl.kernel(
    _deg_body,
    out_type=jax.ShapeDtypeStruct((32, NPAD), f32),
    mesh=_mesh,
    compiler_params=pltpu.CompilerParams(needs_layout_passes=False),
    scratch_types=[
        pltpu.VMEM((CH_E * CHUNK,), i32),
        pltpu.VMEM((NPAD,), f32),
    ],
)


def _prep_body(degp_ref, x_ref, xs0_ref, xs1_ref, dis_ref):
    deg = jnp.sum(degp_ref[...], axis=0) + 1.0
    dis = lax.rsqrt(deg)
    xs = x_ref[...] * dis[:, None]
    xs0_ref[...] = xs[:, :FH]
    xs1_ref[...] = xs[:, FH:]
    dis_ref[...] = jnp.broadcast_to(dis[:, None], dis_ref.shape)


_prep = pl.pallas_call(
    _prep_body,
    grid=(NPAD // MB,),
    in_specs=[
        pl.BlockSpec((32, MB), lambda m: (0, m)),
        pl.BlockSpec((MB, NFEAT), lambda m: (m, 0)),
    ],
    out_specs=[
        pl.BlockSpec((MB, FH), lambda m: (m, 0)),
        pl.BlockSpec((MB, FH), lambda m: (m, 0)),
        pl.BlockSpec((MB, FH), lambda m: (m, 0)),
    ],
    out_shape=[
        jax.ShapeDtypeStruct((NPAD, FH), f32),
        jax.ShapeDtypeStruct((NPAD, FH), f32),
        jax.ShapeDtypeStruct((NPAD, FH), f32),
    ],
)


def _serial_edge_loop(tab_hbm, src_v, dst_v, acc_sh, rows_v, sem, n_chunks):
    def body(j, _):
        pltpu.async_copy(tab_hbm.at[src_v.at[j]], rows_v, sem).wait()
        pltpu.sync_copy(rows_v, acc_sh.at[dst_v.at[j]], add=True)
        return ()

    lax.fori_loop(0, n_chunks, body, ())


def _drain_acc(acc_sh, out_hbm, base, out_base, rows_v):
    for k in range(RPT // CHUNK):
        pltpu.sync_copy(acc_sh.at[pl.ds(base + k * CHUNK, CHUNK)], rows_v)
        pltpu.sync_copy(rows_v, out_hbm.at[pl.ds(out_base + k * CHUNK, CHUNK)])


def _agg1_body(xs0_hbm, xs1_hbm, src_hbm, dst_hbm, out_hbm,
               src_v, dst_v, rows_v, acc_sh, sem):
    c = lax.axis_index("c")
    s = lax.axis_index("s")
    pltpu.sync_copy(src_hbm.at[s], src_v)
    pltpu.sync_copy(dst_hbm.at[s], dst_v)
    base = s * RPT

    def init_and_run(tab_hbm):
        for k in range(RPT // CHUNK):
            pltpu.sync_copy(tab_hbm.at[pl.ds(base + k * CHUNK, CHUNK)], rows_v)
            pltpu.sync_copy(rows_v, acc_sh.at[pl.ds(base + k * CHUNK, CHUNK)])
        plsc.subcore_barrier()
        _serial_edge_loop(tab_hbm, src_v, dst_v, acc_sh, rows_v, sem, CH_C)

    @pl.when(c == 0)
    def _():
        init_and_run(xs0_hbm)

    @pl.when(c == 1)
    def _():
        init_and_run(xs1_hbm)

    plsc.subcore_barrier()
    _drain_acc(acc_sh, out_hbm, base, c * NPAD + base, rows_v)


_agg1_kernel = pl.kernel(
    _agg1_body,
    out_type=jax.ShapeDtypeStruct((2 * NPAD, FH), f32),
    mesh=_mesh,
    scratch_types=[
        pltpu.VMEM((CH_C, CHUNK), i32),
        pltpu.VMEM((CH_C, CHUNK), i32),
        pltpu.VMEM((CHUNK, FH), f32),
        pltpu.VMEM_SHARED((NPAD, FH), f32),
        pltpu.SemaphoreType.DMA,
    ],
)


def _dense_body(a0_ref, a1_ref, dis_ref, w1_ref, b1_ref, w2_ref, ys_ref):
    d = dis_ref[...]
    a0 = a0_ref[...] * d
    a1 = a1_ref[...] * d
    w1 = w1_ref[...]
    h = (
        jnp.dot(a0, w1[:FH], preferred_element_type=f32,
                precision=lax.Precision.HIGHEST)
        + jnp.dot(a1, w1[FH:], preferred_element_type=f32,
                  precision=lax.Precision.HIGHEST)
    )
    h = jnp.maximum(h + b1_ref[...], 0.0)
    y = jnp.dot(h, w2_ref[...], preferred_element_type=f32,
                precision=lax.Precision.HIGHEST)
    ys_ref[...] = y * d[:, :CPAD]


_dense = pl.pallas_call(
    _dense_body,
    grid=(NPAD // MB,),
    in_specs=[
        pl.BlockSpec((MB, FH), lambda m: (m, 0)),
        pl.BlockSpec((MB, FH), lambda m: (m + NPAD // MB, 0)),
        pl.BlockSpec((MB, FH), lambda m: (m, 0)),
        pl.BlockSpec((NFEAT, HIDDEN), lambda m: (0, 0)),
        pl.BlockSpec((1, HIDDEN), lambda m: (0, 0)),
        pl.BlockSpec((HIDDEN, CPAD), lambda m: (0, 0)),
    ],
    out_specs=pl.BlockSpec((MB, CPAD), lambda m: (m, 0)),
    out_shape=jax.ShapeDtypeStruct((NPAD, CPAD), f32),
)


def _agg2_body(ys_hbm, src_hbm, dst_hbm, zero_hbm, out_hbm,
               src_v, dst_v, rows_v, acc_sh, sem):
    c = lax.axis_index("c")
    s = lax.axis_index("s")
    wid = c * 16 + s
    pltpu.sync_copy(src_hbm.at[wid], src_v)
    pltpu.sync_copy(dst_hbm.at[wid], dst_v)
    pltpu.sync_copy(zero_hbm, rows_v)
    base = s * RPT
    for k in range(RPT // CHUNK):
        pltpu.sync_copy(rows_v, acc_sh.at[pl.ds(base + k * CHUNK, CHUNK)])
    plsc.subcore_barrier()
    _serial_edge_loop(ys_hbm, src_v, dst_v, acc_sh, rows_v, sem, CH_E)
    plsc.subcore_barrier()
    _drain_acc(acc_sh, out_hbm, base, c * NPAD + base, rows_v)


_agg2_kernel = pl.kernel(
    _agg2_body,
    out_type=jax.ShapeDtypeStruct((2 * NPAD, CPAD), f32),
    mesh=_mesh,
    compiler_params=pltpu.CompilerParams(use_tc_tiling_on_sc=False),
    scratch_types=[
        pltpu.VMEM((CH_E, CHUNK), i32),
        pltpu.VMEM((CH_E, CHUNK), i32),
        pltpu.VMEM((CHUNK, CPAD), f32),
        pltpu.VMEM_SHARED((NPAD, CPAD), f32),
        pltpu.SemaphoreType.DMA,
    ],
)


def _out_body(a0_ref, a1_ref, ys_ref, dis_ref, b2_ref, o_ref):
    z = (a0_ref[...] + a1_ref[...] + ys_ref[...]) * dis_ref[...][:, :CPAD]
    z = z + b2_ref[...]
    col = lax.broadcasted_iota(i32, z.shape, 1)
    z = jnp.where(col < NCLASS, z, -1e30)
    m = jnp.max(z, axis=1, keepdims=True)
    e = jnp.exp(z - m)
    ssum = jnp.sum(e, axis=1, keepdims=True)
    o_ref[...] = z - m - jnp.log(ssum)


_outk = pl.pallas_call(
    _out_body,
    grid=(NPAD // MB,),
    in_specs=[
        pl.BlockSpec((MB, CPAD), lambda m: (m, 0)),
        pl.BlockSpec((MB, CPAD), lambda m: (m + NPAD // MB, 0)),
        pl.BlockSpec((MB, CPAD), lambda m: (m, 0)),
        pl.BlockSpec((MB, FH), lambda m: (m, 0)),
        pl.BlockSpec((1, CPAD), lambda m: (0, 0)),
    ],
    out_specs=pl.BlockSpec((MB, CPAD), lambda m: (m, 0)),
    out_shape=jax.ShapeDtypeStruct((NPAD, CPAD), f32),
)


def kernel(x, edge_index, W1, b1, W2, b2):
    src = edge_index[0].astype(i32)
    dst = edge_index[1].astype(i32)
    n_pad_edges = EPAD - N_EDGES
    pad_idx = jnp.arange(n_pad_edges, dtype=i32)
    src_pad = jnp.concatenate([src, pad_idx % N_NODES])
    dst_pad = jnp.concatenate([dst, DUMMY + (pad_idx % 128)])
    srcC = src_pad.reshape(16, CH_C, CHUNK)
    dstC = dst_pad.reshape(16, CH_C, CHUNK)
    srcE = src_pad.reshape(32, CH_E, CHUNK)
    dstE = dst_pad.reshape(32, CH_E, CHUNK)
    dstA = dst_pad.reshape(32, CH_E * CHUNK)
    x_pad = jnp.concatenate([x, jnp.zeros((NPAD - N_NODES, NFEAT), f32)])
    w2p = jnp.pad(W2, ((0, 0), (0, CPAD - NCLASS)))
    b2p = jnp.pad(b2, (0, CPAD - NCLASS)).reshape(1, CPAD)
    b1r = b1.reshape(1, HIDDEN)
    zero64 = jnp.zeros((CHUNK, CPAD), f32)

    degp = _deg_kernel(dstA)
    xs0, xs1, dis2d = _prep(degp, x_pad)
    agg = _agg1_kernel(xs0, xs1, srcC, dstC)
    ys = _dense(agg, agg, dis2d, W1, b1r, w2p)
    acc2 = _agg2_kernel(ys, srcE, dstE, zero64)
    o = _outk(acc2, acc2, ys, dis2d, b2p)
    return o[:N_NODES, :NCLASS]


# speedup vs baseline: 3.1358x; 1.2196x over previous
import jax
import jax.numpy as jnp
from jax import lax
from jax.experimental import pallas as pl
from jax.experimental.pallas import tpu as pltpu
from jax.experimental.pallas import tpu_sc as plsc

f32 = jnp.float32
i32 = jnp.int32

N_NODES = 10000
NPAD = 10240
NFEAT = 256
FH = 128
HIDDEN = 512
NCLASS = 40
CPAD = 64
N_EDGES = 160000
CHUNK = 128
CH_C = 80
CH_E = 40
EPAD = 32 * CH_E * CHUNK
DUMMY = N_NODES
RPT = NPAD // 16
MB = 512

_mesh = plsc.VectorSubcoreMesh(
    core_axis_name="c", subcore_axis_name="s", num_cores=2, num_subcores=16
)


def _deg_body(dst_hbm, degp_hbm, dst_v, deg_v):
    c = lax.axis_index("c")
    s = lax.axis_index("s")
    wid = c * 16 + s
    pltpu.sync_copy(dst_hbm.at[wid], dst_v)
    zeros16 = jnp.zeros((16,), f32)

    def zero_body(i, _):
        deg_v[pl.ds(i * 16, 16)] = zeros16
        return ()

    lax.fori_loop(0, NPAD // 16, zero_body, (), unroll=8)
    ones16 = jnp.ones((16,), f32)

    def cnt_body(i, _):
        idx = dst_v[pl.ds(i * 16, 16)]
        plsc.addupdate_scatter(deg_v, [idx], ones16)
        return ()

    lax.fori_loop(0, (CH_E * CHUNK) // 16, cnt_body, (), unroll=8)
    pltpu.sync_copy(deg_v, degp_hbm.at[wid])


_deg_kernel = pl.kernel(
    _deg_body,
    out_type=jax.ShapeDtypeStruct((32, NPAD), f32),
    mesh=_mesh,
    compiler_params=pltpu.CompilerParams(needs_layout_passes=False),
    scratch_types=[
        pltpu.VMEM((CH_E * CHUNK,), i32),
        pltpu.VMEM((NPAD,), f32),
    ],
)


def _prep_body(degp_ref, x_ref, xs0_ref, xs1_ref, dis_ref):
    deg = jnp.sum(degp_ref[...], axis=0) + 1.0
    dis = lax.rsqrt(deg)
    xs = x_ref[...] * dis[:, None]
    xs0_ref[...] = xs[:, :FH]
    xs1_ref[...] = xs[:, FH:]
    dis_ref[...] = jnp.broadcast_to(dis[:, None], dis_ref.shape)


_prep = pl.pallas_call(
    _prep_body,
    grid=(NPAD // MB,),
    in_specs=[
        pl.BlockSpec((32, MB), lambda m: (0, m)),
        pl.BlockSpec((MB, NFEAT), lambda m: (m, 0)),
    ],
    out_specs=[
        pl.BlockSpec((MB, FH), lambda m: (m, 0)),
        pl.BlockSpec((MB, FH), lambda m: (m, 0)),
        pl.BlockSpec((MB, FH), lambda m: (m, 0)),
    ],
    out_shape=[
        jax.ShapeDtypeStruct((NPAD, FH), f32),
        jax.ShapeDtypeStruct((NPAD, FH), f32),
        jax.ShapeDtypeStruct((NPAD, FH), f32),
    ],
)


def _serial_edge_loop(tab_hbm, src_v, dst_v, acc_sh, rows_v, sem, n_chunks):
    def body(j, _):
        pltpu.async_copy(tab_hbm.at[src_v.at[j]], rows_v, sem).wait()
        pltpu.sync_copy(rows_v, acc_sh.at[dst_v.at[j]], add=True)
        return ()

    lax.fori_loop(0, n_chunks, body, ())


def _piped_edge_loop(tab_hbm, src_v, dst_v, acc_sh, r0, r1, s0, s1, n_chunks):
    pltpu.async_copy(tab_hbm.at[src_v.at[0]], r0, s0)

    def body(j, _):
        def step(rb, sb, ro, so):
            @pl.when(j + 1 < n_chunks)
            def _():
                pltpu.async_copy(tab_hbm.at[src_v.at[j + 1]], ro, so)

            pltpu.make_async_copy(tab_hbm.at[src_v.at[j]], rb, sb).wait()
            pltpu.sync_copy(rb, acc_sh.at[dst_v.at[j]], add=True)

        @pl.when(j % 2 == 0)
        def _():
            step(r0, s0, r1, s1)

        @pl.when(j % 2 == 1)
        def _():
            step(r1, s1, r0, s0)

        return ()

    lax.fori_loop(0, n_chunks, body, ())


def _drain_acc(acc_sh, out_hbm, base, out_base, rows_v):
    for k in range(RPT // CHUNK):
        pltpu.sync_copy(acc_sh.at[pl.ds(base + k * CHUNK, CHUNK)], rows_v)
        pltpu.sync_copy(rows_v, out_hbm.at[pl.ds(out_base + k * CHUNK, CHUNK)])


def _agg1_body(xs0_hbm, xs1_hbm, src_hbm, dst_hbm, out_hbm,
               src_v, dst_v, rows_v, acc_sh, sem):
    c = lax.axis_index("c")
    s = lax.axis_index("s")
    pltpu.sync_copy(src_hbm.at[s], src_v)
    pltpu.sync_copy(dst_hbm.at[s], dst_v)
    base = s * RPT

    def init_and_run(tab_hbm):
        for k in range(RPT // CHUNK):
            pltpu.sync_copy(tab_hbm.at[pl.ds(base + k * CHUNK, CHUNK)], rows_v)
            pltpu.sync_copy(rows_v, acc_sh.at[pl.ds(base + k * CHUNK, CHUNK)])
        plsc.subcore_barrier()
        _serial_edge_loop(tab_hbm, src_v, dst_v, acc_sh, rows_v, sem, CH_C)

    @pl.when(c == 0)
    def _():
        init_and_run(xs0_hbm)

    @pl.when(c == 1)
    def _():
        init_and_run(xs1_hbm)

    plsc.subcore_barrier()
    _drain_acc(acc_sh, out_hbm, base, c * NPAD + base, rows_v)


_agg1_kernel = pl.kernel(
    _agg1_body,
    out_type=jax.ShapeDtypeStruct((2 * NPAD, FH), f32),
    mesh=_mesh,
    scratch_types=[
        pltpu.VMEM((CH_C, CHUNK), i32),
        pltpu.VMEM((CH_C, CHUNK), i32),
        pltpu.VMEM((CHUNK, FH), f32),
        pltpu.VMEM_SHARED((NPAD, FH), f32),
        pltpu.SemaphoreType.DMA,
    ],
)


def _dense_body(a0_ref, a1_ref, dis_ref, w1_ref, b1_ref, w2_ref, ys_ref):
    d = dis_ref[...]
    a0 = a0_ref[...] * d
    a1 = a1_ref[...] * d
    w1 = w1_ref[...]
    h = (
        jnp.dot(a0, w1[:FH], preferred_element_type=f32,
                precision=lax.Precision.DEFAULT)
        + jnp.dot(a1, w1[FH:], preferred_element_type=f32,
                  precision=lax.Precision.DEFAULT)
    )
    h = jnp.maximum(h + b1_ref[...], 0.0)
    y = jnp.dot(h, w2_ref[...], preferred_element_type=f32,
                precision=lax.Precision.DEFAULT)
    ys_ref[...] = y * d[:, :CPAD]


_dense = pl.pallas_call(
    _dense_body,
    grid=(NPAD // MB,),
    in_specs=[
        pl.BlockSpec((MB, FH), lambda m: (m, 0)),
        pl.BlockSpec((MB, FH), lambda m: (m + NPAD // MB, 0)),
        pl.BlockSpec((MB, FH), lambda m: (m, 0)),
        pl.BlockSpec((NFEAT, HIDDEN), lambda m: (0, 0)),
        pl.BlockSpec((1, HIDDEN), lambda m: (0, 0)),
        pl.BlockSpec((HIDDEN, CPAD), lambda m: (0, 0)),
    ],
    out_specs=pl.BlockSpec((MB, CPAD), lambda m: (m, 0)),
    out_shape=jax.ShapeDtypeStruct((NPAD, CPAD), f32),
)


def _agg2_body(ys_hbm, src_hbm, dst_hbm, zero_hbm, out_hbm,
               src_v, dst_v, r0, r1, acc_sh, s0, s1):
    c = lax.axis_index("c")
    s = lax.axis_index("s")
    wid = c * 16 + s
    pltpu.sync_copy(src_hbm.at[wid], src_v)
    pltpu.sync_copy(dst_hbm.at[wid], dst_v)
    pltpu.sync_copy(zero_hbm, r0)
    base = s * RPT
    for k in range(RPT // CHUNK):
        pltpu.sync_copy(r0, acc_sh.at[pl.ds(base + k * CHUNK, CHUNK)])
    plsc.subcore_barrier()
    _piped_edge_loop(ys_hbm, src_v, dst_v, acc_sh, r0, r1, s0, s1, CH_E)
    plsc.subcore_barrier()
    _drain_acc(acc_sh, out_hbm, base, c * NPAD + base, r0)


_agg2_kernel = pl.kernel(
    _agg2_body,
    out_type=jax.ShapeDtypeStruct((2 * NPAD, CPAD), f32),
    mesh=_mesh,
    compiler_params=pltpu.CompilerParams(use_tc_tiling_on_sc=False),
    scratch_types=[
        pltpu.VMEM((CH_E, CHUNK), i32),
        pltpu.VMEM((CH_E, CHUNK), i32),
        pltpu.VMEM((CHUNK, CPAD), f32),
        pltpu.VMEM((CHUNK, CPAD), f32),
        pltpu.VMEM_SHARED((NPAD, CPAD), f32),
        pltpu.SemaphoreType.DMA,
        pltpu.SemaphoreType.DMA,
    ],
)


def _out_body(a0_ref, a1_ref, ys_ref, dis_ref, b2_ref, o_ref):
    z = (a0_ref[...] + a1_ref[...] + ys_ref[...]) * dis_ref[...][:, :CPAD]
    z = z + b2_ref[...]
    col = lax.broadcasted_iota(i32, z.shape, 1)
    z = jnp.where(col < NCLASS, z, -1e30)
    m = jnp.max(z, axis=1, keepdims=True)
    e = jnp.exp(z - m)
    ssum = jnp.sum(e, axis=1, keepdims=True)
    o_ref[...] = z - m - jnp.log(ssum)


_outk = pl.pallas_call(
    _out_body,
    grid=(NPAD // MB,),
    in_specs=[
        pl.BlockSpec((MB, CPAD), lambda m: (m, 0)),
        pl.BlockSpec((MB, CPAD), lambda m: (m + NPAD // MB, 0)),
        pl.BlockSpec((MB, CPAD), lambda m: (m, 0)),
        pl.BlockSpec((MB, FH), lambda m: (m, 0)),
        pl.BlockSpec((1, CPAD), lambda m: (0, 0)),
    ],
    out_specs=pl.BlockSpec((MB, CPAD), lambda m: (m, 0)),
    out_shape=jax.ShapeDtypeStruct((NPAD, CPAD), f32),
)


def kernel(x, edge_index, W1, b1, W2, b2):
    src = edge_index[0].astype(i32)
    dst = edge_index[1].astype(i32)
    n_pad_edges = EPAD - N_EDGES
    pad_idx = jnp.arange(n_pad_edges, dtype=i32)
    src_pad = jnp.concatenate([src, pad_idx % N_NODES])
    dst_pad = jnp.concatenate([dst, DUMMY + (pad_idx % 128)])
    srcC = src_pad.reshape(16, CH_C, CHUNK)
    dstC = dst_pad.reshape(16, CH_C, CHUNK)
    srcE = src_pad.reshape(32, CH_E, CHUNK)
    dstE = dst_pad.reshape(32, CH_E, CHUNK)
    dstA = dst_pad.reshape(32, CH_E * CHUNK)
    x_pad = jnp.concatenate([x, jnp.zeros((NPAD - N_NODES, NFEAT), f32)])
    w2p = jnp.pad(W2, ((0, 0), (0, CPAD - NCLASS)))
    b2p = jnp.pad(b2, (0, CPAD - NCLASS)).reshape(1, CPAD)
    b1r = b1.reshape(1, HIDDEN)
    zero64 = jnp.zeros((CHUNK, CPAD), f32)

    degp = _deg_kernel(dstA)
    xs0, xs1, dis2d = _prep(degp, x_pad)
    agg = _agg1_kernel(xs0, xs1, srcC, dstC)
    ys = _dense(agg, agg, dis2d, W1, b1r, w2p)
    acc2 = _agg2_kernel(ys, srcE, dstE, zero64)
    o = _outk(acc2, acc2, ys, dis2d, b2p)
    return o[:N_NODES, :NCLASS]


# speedup vs baseline: 3.7809x; 1.2057x over previous
import jax
import jax.numpy as jnp
from jax import lax
from jax.experimental import pallas as pl
from jax.experimental.pallas import tpu as pltpu
from jax.experimental.pallas import tpu_sc as plsc

f32 = jnp.float32
i32 = jnp.int32

N_NODES = 10000
NPAD = 10240
NFEAT = 256
FH = 128
HIDDEN = 512
NCLASS = 40
CPAD = 64
N_EDGES = 160000
CHUNK = 128
CH_C = 80
CH_E = 40
EPAD = 32 * CH_E * CHUNK
DUMMY = N_NODES
RPT = NPAD // 16
MB = 512

_mesh = plsc.VectorSubcoreMesh(
    core_axis_name="c", subcore_axis_name="s", num_cores=2, num_subcores=16
)


def _deg_body(dst_hbm, degp_hbm, dst_v, deg_v):
    c = lax.axis_index("c")
    s = lax.axis_index("s")
    wid = c * 16 + s
    pltpu.sync_copy(dst_hbm.at[wid], dst_v)
    zeros16 = jnp.zeros((16,), f32)

    def zero_body(i, _):
        deg_v[pl.ds(i * 16, 16)] = zeros16
        return ()

    lax.fori_loop(0, NPAD // 16, zero_body, (), unroll=8)
    ones16 = jnp.ones((16,), f32)

    def cnt_body(i, _):
        idx = dst_v[pl.ds(i * 16, 16)]
        plsc.addupdate_scatter(deg_v, [idx], ones16)
        return ()

    lax.fori_loop(0, (CH_E * CHUNK) // 16, cnt_body, (), unroll=8)
    pltpu.sync_copy(deg_v, degp_hbm.at[wid])


_deg_kernel = pl.kernel(
    _deg_body,
    out_type=jax.ShapeDtypeStruct((32, NPAD), f32),
    mesh=_mesh,
    compiler_params=pltpu.CompilerParams(needs_layout_passes=False),
    scratch_types=[
        pltpu.VMEM((CH_E * CHUNK,), i32),
        pltpu.VMEM((NPAD,), f32),
    ],
)


def _prep_body(degp_ref, x_ref, xs0_ref, xs1_ref, dis_ref):
    deg = jnp.sum(degp_ref[...], axis=0) + 1.0
    dis = lax.rsqrt(deg)
    xs = x_ref[...] * dis[:, None]
    xs0_ref[...] = xs[:, :FH]
    xs1_ref[...] = xs[:, FH:]
    dis_ref[...] = jnp.broadcast_to(dis[:, None], dis_ref.shape)


_prep = pl.pallas_call(
    _prep_body,
    grid=(NPAD // MB,),
    in_specs=[
        pl.BlockSpec((32, MB), lambda m: (0, m)),
        pl.BlockSpec((MB, NFEAT), lambda m: (m, 0)),
    ],
    out_specs=[
        pl.BlockSpec((MB, FH), lambda m: (m, 0)),
        pl.BlockSpec((MB, FH), lambda m: (m, 0)),
        pl.BlockSpec((MB, FH), lambda m: (m, 0)),
    ],
    out_shape=[
        jax.ShapeDtypeStruct((NPAD, FH), f32),
        jax.ShapeDtypeStruct((NPAD, FH), f32),
        jax.ShapeDtypeStruct((NPAD, FH), f32),
    ],
)


def _serial_edge_loop(tab_hbm, src_v, dst_v, acc_sh, rows_v, sem, n_chunks):
    def body(j, _):
        pltpu.async_copy(tab_hbm.at[src_v.at[j]], rows_v, sem).wait()
        pltpu.sync_copy(rows_v, acc_sh.at[dst_v.at[j]], add=True)
        return ()

    lax.fori_loop(0, n_chunks, body, ())


def _piped_edge_loop(tab_hbm, src_v, dst_v, acc_sh, r0, r1, s0, s1, n_chunks):
    pltpu.async_copy(tab_hbm.at[src_v.at[0]], r0, s0)

    def body(j, _):
        def step(rb, sb, ro, so):
            @pl.when(j + 1 < n_chunks)
            def _():
                pltpu.async_copy(tab_hbm.at[src_v.at[j + 1]], ro, so)

            pltpu.make_async_copy(tab_hbm.at[src_v.at[j]], rb, sb).wait()
            pltpu.sync_copy(rb, acc_sh.at[dst_v.at[j]], add=True)

        @pl.when(j % 2 == 0)
        def _():
            step(r0, s0, r1, s1)

        @pl.when(j % 2 == 1)
        def _():
            step(r1, s1, r0, s0)

        return ()

    lax.fori_loop(0, n_chunks, body, ())


def _drain_acc(acc_sh, out_hbm, base, out_base, rows_v):
    for k in range(RPT // CHUNK):
        pltpu.sync_copy(acc_sh.at[pl.ds(base + k * CHUNK, CHUNK)], rows_v)
        pltpu.sync_copy(rows_v, out_hbm.at[pl.ds(out_base + k * CHUNK, CHUNK)])


GC = 8


def _agg1_body(xs0_hbm, xs1_hbm, src_hbm, dst_hbm, out_hbm,
               si0, si1, di0, di1, r0, r1, acc_sh, gs0, gs1, s0, s1):
    c = lax.axis_index("c")
    s = lax.axis_index("s")
    base = s * RPT

    def init_and_run(tab):
        for k in range(RPT // CHUNK):
            pltpu.sync_copy(tab.at[pl.ds(base + k * CHUNK, CHUNK)], r0)
            pltpu.sync_copy(r0, acc_sh.at[pl.ds(base + k * CHUNK, CHUNK)])
        plsc.subcore_barrier()
        ng = CH_C // GC
        pltpu.async_copy(src_hbm.at[s, 0], si0, gs0)
        pltpu.async_copy(dst_hbm.at[s, 0], di0, gs0)

        def gbody(g, _):
            def run_group(si, di, gsem, sin, din, gsemn):
                pltpu.make_async_copy(src_hbm.at[s, g], si, gsem).wait()
                pltpu.make_async_copy(dst_hbm.at[s, g], di, gsem).wait()

                @pl.when(g + 1 < ng)
                def _():
                    pltpu.async_copy(src_hbm.at[s, g + 1], sin, gsemn)
                    pltpu.async_copy(dst_hbm.at[s, g + 1], din, gsemn)

                pltpu.async_copy(tab.at[si.at[0]], r0, s0)
                for k in range(GC):
                    rb, sb = (r0, s0) if k % 2 == 0 else (r1, s1)
                    ro, so = (r1, s1) if k % 2 == 0 else (r0, s0)
                    if k + 1 < GC:
                        pltpu.async_copy(tab.at[si.at[k + 1]], ro, so)
                    pltpu.make_async_copy(tab.at[si.at[k]], rb, sb).wait()
                    pltpu.sync_copy(rb, acc_sh.at[di.at[k]], add=True)

            @pl.when(g % 2 == 0)
            def _():
                run_group(si0, di0, gs0, si1, di1, gs1)

            @pl.when(g % 2 == 1)
            def _():
                run_group(si1, di1, gs1, si0, di0, gs0)

            return ()

        lax.fori_loop(0, ng, gbody, ())

    @pl.when(c == 0)
    def _():
        init_and_run(xs0_hbm)

    @pl.when(c == 1)
    def _():
        init_and_run(xs1_hbm)

    plsc.subcore_barrier()
    _drain_acc(acc_sh, out_hbm, base, c * NPAD + base, r0)


_agg1_kernel = pl.kernel(
    _agg1_body,
    out_type=jax.ShapeDtypeStruct((2 * NPAD, FH), f32),
    mesh=_mesh,
    scratch_types=[
        pltpu.VMEM((GC, CHUNK), i32),
        pltpu.VMEM((GC, CHUNK), i32),
        pltpu.VMEM((GC, CHUNK), i32),
        pltpu.VMEM((GC, CHUNK), i32),
        pltpu.VMEM((CHUNK, FH), f32),
        pltpu.VMEM((CHUNK, FH), f32),
        pltpu.VMEM_SHARED((NPAD, FH), f32),
        pltpu.SemaphoreType.DMA,
        pltpu.SemaphoreType.DMA,
        pltpu.SemaphoreType.DMA,
        pltpu.SemaphoreType.DMA,
    ],
)


def _dense_body(a0_ref, a1_ref, dis_ref, w1_ref, b1_ref, w2_ref, ys_ref):
    d = dis_ref[...]
    a0 = a0_ref[...] * d
    a1 = a1_ref[...] * d
    w1 = w1_ref[...]
    h = (
        jnp.dot(a0, w1[:FH], preferred_element_type=f32,
                precision=lax.Precision.DEFAULT)
        + jnp.dot(a1, w1[FH:], preferred_element_type=f32,
                  precision=lax.Precision.DEFAULT)
    )
    h = jnp.maximum(h + b1_ref[...], 0.0)
    y = jnp.dot(h, w2_ref[...], preferred_element_type=f32,
                precision=lax.Precision.DEFAULT)
    ys_ref[...] = y * d[:, :CPAD]


_dense = pl.pallas_call(
    _dense_body,
    grid=(NPAD // MB,),
    in_specs=[
        pl.BlockSpec((MB, FH), lambda m: (m, 0)),
        pl.BlockSpec((MB, FH), lambda m: (m + NPAD // MB, 0)),
        pl.BlockSpec((MB, FH), lambda m: (m, 0)),
        pl.BlockSpec((NFEAT, HIDDEN), lambda m: (0, 0)),
        pl.BlockSpec((1, HIDDEN), lambda m: (0, 0)),
        pl.BlockSpec((HIDDEN, CPAD), lambda m: (0, 0)),
    ],
    out_specs=pl.BlockSpec((MB, CPAD), lambda m: (m, 0)),
    out_shape=jax.ShapeDtypeStruct((NPAD, CPAD), f32),
)


def _agg2_body(ys_hbm, src_hbm, dst_hbm, zero_hbm, out_hbm,
               src_v, dst_v, r0, r1, acc_sh, s0, s1):
    c = lax.axis_index("c")
    s = lax.axis_index("s")
    wid = c * 16 + s
    pltpu.sync_copy(src_hbm.at[wid], src_v)
    pltpu.sync_copy(dst_hbm.at[wid], dst_v)
    pltpu.sync_copy(zero_hbm, r0)
    base = s * RPT
    for k in range(RPT // CHUNK):
        pltpu.sync_copy(r0, acc_sh.at[pl.ds(base + k * CHUNK, CHUNK)])
    plsc.subcore_barrier()
    _piped_edge_loop(ys_hbm, src_v, dst_v, acc_sh, r0, r1, s0, s1, CH_E)
    plsc.subcore_barrier()
    _drain_acc(acc_sh, out_hbm, base, c * NPAD + base, r0)


_agg2_kernel = pl.kernel(
    _agg2_body,
    out_type=jax.ShapeDtypeStruct((2 * NPAD, CPAD), f32),
    mesh=_mesh,
    compiler_params=pltpu.CompilerParams(use_tc_tiling_on_sc=False),
    scratch_types=[
        pltpu.VMEM((CH_E, CHUNK), i32),
        pltpu.VMEM((CH_E, CHUNK), i32),
        pltpu.VMEM((CHUNK, CPAD), f32),
        pltpu.VMEM((CHUNK, CPAD), f32),
        pltpu.VMEM_SHARED((NPAD, CPAD), f32),
        pltpu.SemaphoreType.DMA,
        pltpu.SemaphoreType.DMA,
    ],
)


def _out_body(a0_ref, a1_ref, ys_ref, dis_ref, b2_ref, o_ref):
    z = (a0_ref[...] + a1_ref[...] + ys_ref[...]) * dis_ref[...][:, :CPAD]
    z = z + b2_ref[...]
    col = lax.broadcasted_iota(i32, z.shape, 1)
    z = jnp.where(col < NCLASS, z, -1e30)
    m = jnp.max(z, axis=1, keepdims=True)
    e = jnp.exp(z - m)
    ssum = jnp.sum(e, axis=1, keepdims=True)
    o_ref[...] = z - m - jnp.log(ssum)


_outk = pl.pallas_call(
    _out_body,
    grid=(NPAD // MB,),
    in_specs=[
        pl.BlockSpec((MB, CPAD), lambda m: (m, 0)),
        pl.BlockSpec((MB, CPAD), lambda m: (m + NPAD // MB, 0)),
        pl.BlockSpec((MB, CPAD), lambda m: (m, 0)),
        pl.BlockSpec((MB, FH), lambda m: (m, 0)),
        pl.BlockSpec((1, CPAD), lambda m: (0, 0)),
    ],
    out_specs=pl.BlockSpec((MB, CPAD), lambda m: (m, 0)),
    out_shape=jax.ShapeDtypeStruct((NPAD, CPAD), f32),
)


def kernel(x, edge_index, W1, b1, W2, b2):
    src = edge_index[0].astype(i32)
    dst = edge_index[1].astype(i32)
    n_pad_edges = EPAD - N_EDGES
    pad_idx = jnp.arange(n_pad_edges, dtype=i32)
    src_pad = jnp.concatenate([src, pad_idx % N_NODES])
    dst_pad = jnp.concatenate([dst, DUMMY + (pad_idx % 128)])
    srcC = src_pad.reshape(16, CH_C // GC, GC, CHUNK)
    dstC = dst_pad.reshape(16, CH_C // GC, GC, CHUNK)
    srcE = src_pad.reshape(32, CH_E, CHUNK)
    dstE = dst_pad.reshape(32, CH_E, CHUNK)
    dstA = dst_pad.reshape(32, CH_E * CHUNK)
    x_pad = jnp.concatenate([x, jnp.zeros((NPAD - N_NODES, NFEAT), f32)])
    w2p = jnp.pad(W2, ((0, 0), (0, CPAD - NCLASS)))
    b2p = jnp.pad(b2, (0, CPAD - NCLASS)).reshape(1, CPAD)
    b1r = b1.reshape(1, HIDDEN)
    zero64 = jnp.zeros((CHUNK, CPAD), f32)

    degp = _deg_kernel(dstA)
    xs0, xs1, dis2d = _prep(degp, x_pad)
    agg = _agg1_kernel(xs0, xs1, srcC, dstC)
    ys = _dense(agg, agg, dis2d, W1, b1r, w2p)
    acc2 = _agg2_kernel(ys, srcE, dstE, zero64)
    o = _outk(acc2, acc2, ys, dis2d, b2p)
    return o[:N_NODES, :NCLASS]


# speedup vs baseline: 3.8947x; 1.0301x over previous
import jax
import jax.numpy as jnp
from jax import lax
from jax.experimental import pallas as pl
from jax.experimental.pallas import tpu as pltpu
from jax.experimental.pallas import tpu_sc as plsc

f32 = jnp.float32
i32 = jnp.int32

N_NODES = 10000
NPAD = 10240
NFEAT = 256
FH = 128
HIDDEN = 512
NCLASS = 40
CPAD = 64
N_EDGES = 160000
CHUNK = 128
CH_C = 80
CH_E = 40
EPAD = 32 * CH_E * CHUNK
DUMMY = N_NODES
RPT = NPAD // 16
MB = 512

_mesh = plsc.VectorSubcoreMesh(
    core_axis_name="c", subcore_axis_name="s", num_cores=2, num_subcores=16
)


def _deg_body(dst_hbm, degp_hbm, dst_v, deg_v):
    c = lax.axis_index("c")
    s = lax.axis_index("s")
    wid = c * 16 + s
    pltpu.sync_copy(dst_hbm.at[wid], dst_v)
    zeros16 = jnp.zeros((16,), f32)

    def zero_body(i, _):
        deg_v[pl.ds(i * 16, 16)] = zeros16
        return ()

    lax.fori_loop(0, NPAD // 16, zero_body, (), unroll=8)
    ones16 = jnp.ones((16,), f32)

    def cnt_body(i, _):
        idx = dst_v[pl.ds(i * 16, 16)]
        plsc.addupdate_scatter(deg_v, [idx], ones16)
        return ()

    lax.fori_loop(0, (CH_E * CHUNK) // 16, cnt_body, (), unroll=8)
    pltpu.sync_copy(deg_v, degp_hbm.at[wid])


_deg_kernel = pl.kernel(
    _deg_body,
    out_type=jax.ShapeDtypeStruct((32, NPAD), f32),
    mesh=_mesh,
    compiler_params=pltpu.CompilerParams(needs_layout_passes=False),
    scratch_types=[
        pltpu.VMEM((CH_E * CHUNK,), i32),
        pltpu.VMEM((NPAD,), f32),
    ],
)


def _prep_body(degp_ref, x_ref, xs0_ref, xs1_ref, dis_ref):
    deg = jnp.sum(degp_ref[...], axis=0) + 1.0
    dis = lax.rsqrt(deg)
    xs = x_ref[...] * dis[:, None]
    xs0_ref[...] = xs[:, :FH]
    xs1_ref[...] = xs[:, FH:]
    dis_ref[...] = jnp.broadcast_to(dis[:, None], dis_ref.shape)


_prep = pl.pallas_call(
    _prep_body,
    grid=(NPAD // MB,),
    in_specs=[
        pl.BlockSpec((32, MB), lambda m: (0, m)),
        pl.BlockSpec((MB, NFEAT), lambda m: (m, 0)),
    ],
    out_specs=[
        pl.BlockSpec((MB, FH), lambda m: (m, 0)),
        pl.BlockSpec((MB, FH), lambda m: (m, 0)),
        pl.BlockSpec((MB, FH), lambda m: (m, 0)),
    ],
    out_shape=[
        jax.ShapeDtypeStruct((NPAD, FH), f32),
        jax.ShapeDtypeStruct((NPAD, FH), f32),
        jax.ShapeDtypeStruct((NPAD, FH), f32),
    ],
)


def _serial_edge_loop(tab_hbm, src_v, dst_v, acc_sh, rows_v, sem, n_chunks):
    def body(j, _):
        pltpu.async_copy(tab_hbm.at[src_v.at[j]], rows_v, sem).wait()
        pltpu.sync_copy(rows_v, acc_sh.at[dst_v.at[j]], add=True)
        return ()

    lax.fori_loop(0, n_chunks, body, ())


def _piped_edge_loop(tab_hbm, src_v, dst_v, acc_sh, bufs, sems, n_chunks):
    nb = len(bufs)
    for t in range(min(nb, n_chunks)):
        pltpu.async_copy(tab_hbm.at[src_v.at[t]], bufs[t], sems[t])

    def body(j, _):
        for p in range(nb):
            @pl.when(j % nb == p)
            def _(p=p):
                pltpu.make_async_copy(
                    tab_hbm.at[src_v.at[j]], bufs[p], sems[p]).wait()
                pltpu.sync_copy(bufs[p], acc_sh.at[dst_v.at[j]], add=True)

                @pl.when(j + nb < n_chunks)
                def _():
                    pltpu.async_copy(
                        tab_hbm.at[src_v.at[j + nb]], bufs[p], sems[p])

        return ()

    lax.fori_loop(0, n_chunks, body, ())


def _drain_acc(acc_sh, out_hbm, base, out_base, rows_v):
    for k in range(RPT // CHUNK):
        pltpu.sync_copy(acc_sh.at[pl.ds(base + k * CHUNK, CHUNK)], rows_v)
        pltpu.sync_copy(rows_v, out_hbm.at[pl.ds(out_base + k * CHUNK, CHUNK)])


GC = 8


def _agg1_body(xs0_hbm, xs1_hbm, src_hbm, dst_hbm, out_hbm,
               si0, si1, di0, di1, r0, r1, acc_sh, gs0, gs1, s0, s1):
    c = lax.axis_index("c")
    s = lax.axis_index("s")
    base = s * RPT

    def init_and_run(tab):
        for k in range(RPT // CHUNK):
            pltpu.sync_copy(tab.at[pl.ds(base + k * CHUNK, CHUNK)], r0)
            pltpu.sync_copy(r0, acc_sh.at[pl.ds(base + k * CHUNK, CHUNK)])
        plsc.subcore_barrier()
        ng = CH_C // GC
        pltpu.async_copy(src_hbm.at[s, 0], si0, gs0)
        pltpu.async_copy(dst_hbm.at[s, 0], di0, gs0)

        def gbody(g, _):
            def run_group(si, di, gsem, sin, din, gsemn):
                pltpu.make_async_copy(src_hbm.at[s, g], si, gsem).wait()
                pltpu.make_async_copy(dst_hbm.at[s, g], di, gsem).wait()

                @pl.when(g + 1 < ng)
                def _():
                    pltpu.async_copy(src_hbm.at[s, g + 1], sin, gsemn)
                    pltpu.async_copy(dst_hbm.at[s, g + 1], din, gsemn)

                pltpu.async_copy(tab.at[si.at[0]], r0, s0)
                for k in range(GC):
                    rb, sb = (r0, s0) if k % 2 == 0 else (r1, s1)
                    ro, so = (r1, s1) if k % 2 == 0 else (r0, s0)
                    if k + 1 < GC:
                        pltpu.async_copy(tab.at[si.at[k + 1]], ro, so)
                    pltpu.make_async_copy(tab.at[si.at[k]], rb, sb).wait()
                    pltpu.sync_copy(rb, acc_sh.at[di.at[k]], add=True)

            @pl.when(g % 2 == 0)
            def _():
                run_group(si0, di0, gs0, si1, di1, gs1)

            @pl.when(g % 2 == 1)
            def _():
                run_group(si1, di1, gs1, si0, di0, gs0)

            return ()

        lax.fori_loop(0, ng, gbody, ())

    @pl.when(c == 0)
    def _():
        init_and_run(xs0_hbm)

    @pl.when(c == 1)
    def _():
        init_and_run(xs1_hbm)

    plsc.subcore_barrier()
    _drain_acc(acc_sh, out_hbm, base, c * NPAD + base, r0)


_agg1_kernel = pl.kernel(
    _agg1_body,
    out_type=jax.ShapeDtypeStruct((2 * NPAD, FH), f32),
    mesh=_mesh,
    scratch_types=[
        pltpu.VMEM((GC, CHUNK), i32),
        pltpu.VMEM((GC, CHUNK), i32),
        pltpu.VMEM((GC, CHUNK), i32),
        pltpu.VMEM((GC, CHUNK), i32),
        pltpu.VMEM((CHUNK, FH), f32),
        pltpu.VMEM((CHUNK, FH), f32),
        pltpu.VMEM_SHARED((NPAD, FH), f32),
        pltpu.SemaphoreType.DMA,
        pltpu.SemaphoreType.DMA,
        pltpu.SemaphoreType.DMA,
        pltpu.SemaphoreType.DMA,
    ],
)


def _dense_body(a0_ref, a1_ref, dis_ref, w1_ref, b1_ref, w2_ref, ys_ref):
    d = dis_ref[...]
    a0 = a0_ref[...] * d
    a1 = a1_ref[...] * d
    w1 = w1_ref[...]
    h = (
        jnp.dot(a0, w1[:FH], preferred_element_type=f32,
                precision=lax.Precision.DEFAULT)
        + jnp.dot(a1, w1[FH:], preferred_element_type=f32,
                  precision=lax.Precision.DEFAULT)
    )
    h = jnp.maximum(h + b1_ref[...], 0.0)
    y = jnp.dot(h, w2_ref[...], preferred_element_type=f32,
                precision=lax.Precision.DEFAULT)
    ys_ref[...] = y * d[:, :CPAD]


_dense = pl.pallas_call(
    _dense_body,
    grid=(NPAD // MB,),
    in_specs=[
        pl.BlockSpec((MB, FH), lambda m: (m, 0)),
        pl.BlockSpec((MB, FH), lambda m: (m + NPAD // MB, 0)),
        pl.BlockSpec((MB, FH), lambda m: (m, 0)),
        pl.BlockSpec((NFEAT, HIDDEN), lambda m: (0, 0)),
        pl.BlockSpec((1, HIDDEN), lambda m: (0, 0)),
        pl.BlockSpec((HIDDEN, CPAD), lambda m: (0, 0)),
    ],
    out_specs=pl.BlockSpec((MB, CPAD), lambda m: (m, 0)),
    out_shape=jax.ShapeDtypeStruct((NPAD, CPAD), f32),
)


def _agg2_body(ys_hbm, src_hbm, dst_hbm, zero_hbm, out_hbm,
               src_v, dst_v, r0, r1, r2, r3, acc_sh, s0, s1, s2, s3):
    c = lax.axis_index("c")
    s = lax.axis_index("s")
    wid = c * 16 + s
    pltpu.sync_copy(src_hbm.at[wid], src_v)
    pltpu.sync_copy(dst_hbm.at[wid], dst_v)
    pltpu.sync_copy(zero_hbm, r0)
    base = s * RPT
    for k in range(RPT // CHUNK):
        pltpu.sync_copy(r0, acc_sh.at[pl.ds(base + k * CHUNK, CHUNK)])
    plsc.subcore_barrier()
    _piped_edge_loop(ys_hbm, src_v, dst_v, acc_sh,
                     (r0, r1, r2, r3), (s0, s1, s2, s3), CH_E)
    plsc.subcore_barrier()
    _drain_acc(acc_sh, out_hbm, base, c * NPAD + base, r0)


_agg2_kernel = pl.kernel(
    _agg2_body,
    out_type=jax.ShapeDtypeStruct((2 * NPAD, CPAD), f32),
    mesh=_mesh,
    compiler_params=pltpu.CompilerParams(use_tc_tiling_on_sc=False),
    scratch_types=[
        pltpu.VMEM((CH_E, CHUNK), i32),
        pltpu.VMEM((CH_E, CHUNK), i32),
        pltpu.VMEM((CHUNK, CPAD), f32),
        pltpu.VMEM((CHUNK, CPAD), f32),
        pltpu.VMEM((CHUNK, CPAD), f32),
        pltpu.VMEM((CHUNK, CPAD), f32),
        pltpu.VMEM_SHARED((NPAD, CPAD), f32),
        pltpu.SemaphoreType.DMA,
        pltpu.SemaphoreType.DMA,
        pltpu.SemaphoreType.DMA,
        pltpu.SemaphoreType.DMA,
    ],
)


def _out_body(a0_ref, a1_ref, ys_ref, dis_ref, b2_ref, o_ref):
    z = (a0_ref[...] + a1_ref[...] + ys_ref[...]) * dis_ref[...][:, :CPAD]
    z = z + b2_ref[...]
    col = lax.broadcasted_iota(i32, z.shape, 1)
    z = jnp.where(col < NCLASS, z, -1e30)
    m = jnp.max(z, axis=1, keepdims=True)
    e = jnp.exp(z - m)
    ssum = jnp.sum(e, axis=1, keepdims=True)
    o_ref[...] = z - m - jnp.log(ssum)


_outk = pl.pallas_call(
    _out_body,
    grid=(NPAD // MB,),
    in_specs=[
        pl.BlockSpec((MB, CPAD), lambda m: (m, 0)),
        pl.BlockSpec((MB, CPAD), lambda m: (m + NPAD // MB, 0)),
        pl.BlockSpec((MB, CPAD), lambda m: (m, 0)),
        pl.BlockSpec((MB, FH), lambda m: (m, 0)),
        pl.BlockSpec((1, CPAD), lambda m: (0, 0)),
    ],
    out_specs=pl.BlockSpec((MB, CPAD), lambda m: (m, 0)),
    out_shape=jax.ShapeDtypeStruct((NPAD, CPAD), f32),
)


def kernel(x, edge_index, W1, b1, W2, b2):
    src = edge_index[0].astype(i32)
    dst = edge_index[1].astype(i32)
    n_pad_edges = EPAD - N_EDGES
    pad_idx = jnp.arange(n_pad_edges, dtype=i32)
    src_pad = jnp.concatenate([src, pad_idx % N_NODES])
    dst_pad = jnp.concatenate([dst, DUMMY + (pad_idx % 128)])
    srcC = src_pad.reshape(16, CH_C // GC, GC, CHUNK)
    dstC = dst_pad.reshape(16, CH_C // GC, GC, CHUNK)
    srcE = src_pad.reshape(32, CH_E, CHUNK)
    dstE = dst_pad.reshape(32, CH_E, CHUNK)
    dstA = dst_pad.reshape(32, CH_E * CHUNK)
    x_pad = jnp.concatenate([x, jnp.zeros((NPAD - N_NODES, NFEAT), f32)])
    w2p = jnp.pad(W2, ((0, 0), (0, CPAD - NCLASS)))
    b2p = jnp.pad(b2, (0, CPAD - NCLASS)).reshape(1, CPAD)
    b1r = b1.reshape(1, HIDDEN)
    zero64 = jnp.zeros((CHUNK, CPAD), f32)

    degp = _deg_kernel(dstA)
    xs0, xs1, dis2d = _prep(degp, x_pad)
    agg = _agg1_kernel(xs0, xs1, srcC, dstC)
    ys = _dense(agg, agg, dis2d, W1, b1r, w2p)
    acc2 = _agg2_kernel(ys, srcE, dstE, zero64)
    o = _outk(acc2, acc2, ys, dis2d, b2p)
    return o[:N_NODES, :NCLASS]


# speedup vs baseline: 3.9885x; 1.0241x over previous
import jax
import jax.numpy as jnp
from jax import lax
from jax.experimental import pallas as pl
from jax.experimental.pallas import tpu as pltpu
from jax.experimental.pallas import tpu_sc as plsc

f32 = jnp.float32
i32 = jnp.int32

N_NODES = 10000
NPAD = 10240
NFEAT = 256
FH = 128
HIDDEN = 512
NCLASS = 40
CPAD = 64
N_EDGES = 160000
CHUNK = 128
CH_C = 80
CH_E = 40
EPAD = 32 * CH_E * CHUNK
DUMMY = N_NODES
RPT = NPAD // 16
MB = 512

_mesh = plsc.VectorSubcoreMesh(
    core_axis_name="c", subcore_axis_name="s", num_cores=2, num_subcores=16
)


def _deg_body(dst_hbm, degp_hbm, dst_v, deg_v):
    c = lax.axis_index("c")
    s = lax.axis_index("s")
    wid = c * 16 + s
    pltpu.sync_copy(dst_hbm.at[wid], dst_v)
    zeros16 = jnp.zeros((16,), f32)

    def zero_body(i, _):
        deg_v[pl.ds(i * 16, 16)] = zeros16
        return ()

    lax.fori_loop(0, NPAD // 16, zero_body, (), unroll=8)
    ones16 = jnp.ones((16,), f32)

    def cnt_body(i, _):
        idx = dst_v[pl.ds(i * 16, 16)]
        plsc.addupdate_scatter(deg_v, [idx], ones16)
        return ()

    lax.fori_loop(0, (CH_E * CHUNK) // 16, cnt_body, (), unroll=8)
    pltpu.sync_copy(deg_v, degp_hbm.at[wid])


_deg_kernel = pl.kernel(
    _deg_body,
    out_type=jax.ShapeDtypeStruct((32, NPAD), f32),
    mesh=_mesh,
    compiler_params=pltpu.CompilerParams(needs_layout_passes=False),
    scratch_types=[
        pltpu.VMEM((CH_E * CHUNK,), i32),
        pltpu.VMEM((NPAD,), f32),
    ],
)


def _prep_body(degp_ref, x_ref, xs0_ref, xs1_ref, dis_ref):
    deg = jnp.sum(degp_ref[...], axis=0) + 1.0
    dis = lax.rsqrt(deg)
    xs = x_ref[...] * dis[:, None]
    xs0_ref[...] = xs[:, :FH]
    xs1_ref[...] = xs[:, FH:]
    dis_ref[...] = jnp.broadcast_to(dis[:, None], dis_ref.shape)


_prep = pl.pallas_call(
    _prep_body,
    grid=(NPAD // MB,),
    in_specs=[
        pl.BlockSpec((32, MB), lambda m: (0, m)),
        pl.BlockSpec((MB, NFEAT), lambda m: (m, 0)),
    ],
    out_specs=[
        pl.BlockSpec((MB, FH), lambda m: (m, 0)),
        pl.BlockSpec((MB, FH), lambda m: (m, 0)),
        pl.BlockSpec((MB, FH), lambda m: (m, 0)),
    ],
    out_shape=[
        jax.ShapeDtypeStruct((NPAD, FH), f32),
        jax.ShapeDtypeStruct((NPAD, FH), f32),
        jax.ShapeDtypeStruct((NPAD, FH), f32),
    ],
)


def _serial_edge_loop(tab_hbm, src_v, dst_v, acc_sh, rows_v, sem, n_chunks):
    def body(j, _):
        pltpu.async_copy(tab_hbm.at[src_v.at[j]], rows_v, sem).wait()
        pltpu.sync_copy(rows_v, acc_sh.at[dst_v.at[j]], add=True)
        return ()

    lax.fori_loop(0, n_chunks, body, ())


def _piped_edge_loop(tab_hbm, src_v, dst_v, acc_sh, bufs, sems, n_chunks):
    nb = len(bufs)
    for t in range(min(nb, n_chunks)):
        pltpu.async_copy(tab_hbm.at[src_v.at[t]], bufs[t], sems[t])

    def body(j, _):
        for p in range(nb):
            @pl.when(j % nb == p)
            def _(p=p):
                pltpu.make_async_copy(
                    tab_hbm.at[src_v.at[j]], bufs[p], sems[p]).wait()
                pltpu.sync_copy(bufs[p], acc_sh.at[dst_v.at[j]], add=True)

                @pl.when(j + nb < n_chunks)
                def _():
                    pltpu.async_copy(
                        tab_hbm.at[src_v.at[j + nb]], bufs[p], sems[p])

        return ()

    lax.fori_loop(0, n_chunks, body, ())


def _drain_acc(acc_sh, out_hbm, base, out_base, rows_v):
    for k in range(RPT // CHUNK):
        pltpu.sync_copy(acc_sh.at[pl.ds(base + k * CHUNK, CHUNK)], rows_v)
        pltpu.sync_copy(rows_v, out_hbm.at[pl.ds(out_base + k * CHUNK, CHUNK)])


CHC = 64
GCC = 16
NGC = (EPAD // 16) // (GCC * CHC)
NBC = 4


def _agg1_body(xs0_hbm, xs1_hbm, src_hbm, dst_hbm, out_hbm,
               si0, si1, di0, di1, r0, r1, r2, r3,
               acc_sh, gs0, gs1, s0, s1, s2, s3):
    bufs = (r0, r1, r2, r3)
    sems = (s0, s1, s2, s3)
    c = lax.axis_index("c")
    s = lax.axis_index("s")
    base = s * RPT

    def init_and_run(tab):
        for k in range(RPT // CHC):
            pltpu.sync_copy(tab.at[pl.ds(base + k * CHC, CHC)], r0)
            pltpu.sync_copy(r0, acc_sh.at[pl.ds(base + k * CHC, CHC)])
        plsc.subcore_barrier()
        pltpu.async_copy(src_hbm.at[s, 0], si0, gs0)
        pltpu.async_copy(dst_hbm.at[s, 0], di0, gs0)

        def gbody(g, _):
            def run_group(si, di, gsem, sin, din, gsemn):
                pltpu.make_async_copy(src_hbm.at[s, g], si, gsem).wait()
                pltpu.make_async_copy(dst_hbm.at[s, g], di, gsem).wait()

                @pl.when(g + 1 < NGC)
                def _():
                    pltpu.async_copy(src_hbm.at[s, g + 1], sin, gsemn)
                    pltpu.async_copy(dst_hbm.at[s, g + 1], din, gsemn)

                for t in range(NBC):
                    pltpu.async_copy(tab.at[si.at[t]], bufs[t], sems[t])
                for k in range(GCC):
                    p = k % NBC
                    pltpu.make_async_copy(
                        tab.at[si.at[k]], bufs[p], sems[p]).wait()
                    pltpu.sync_copy(bufs[p], acc_sh.at[di.at[k]], add=True)
                    if k + NBC < GCC:
                        pltpu.async_copy(
                            tab.at[si.at[k + NBC]], bufs[p], sems[p])

            @pl.when(g % 2 == 0)
            def _():
                run_group(si0, di0, gs0, si1, di1, gs1)

            @pl.when(g % 2 == 1)
            def _():
                run_group(si1, di1, gs1, si0, di0, gs0)

            return ()

        lax.fori_loop(0, NGC, gbody, ())

    @pl.when(c == 0)
    def _():
        init_and_run(xs0_hbm)

    @pl.when(c == 1)
    def _():
        init_and_run(xs1_hbm)

    plsc.subcore_barrier()
    for k in range(RPT // CHC):
        pltpu.sync_copy(acc_sh.at[pl.ds(base + k * CHC, CHC)], r0)
        pltpu.sync_copy(
            r0, out_hbm.at[pl.ds(c * NPAD + base + k * CHC, CHC)])


_agg1_kernel = pl.kernel(
    _agg1_body,
    out_type=jax.ShapeDtypeStruct((2 * NPAD, FH), f32),
    mesh=_mesh,
    compiler_params=pltpu.CompilerParams(use_tc_tiling_on_sc=False),
    scratch_types=[
        pltpu.VMEM((GCC, CHC), i32),
        pltpu.VMEM((GCC, CHC), i32),
        pltpu.VMEM((GCC, CHC), i32),
        pltpu.VMEM((GCC, CHC), i32),
        pltpu.VMEM((CHC, FH), f32),
        pltpu.VMEM((CHC, FH), f32),
        pltpu.VMEM((CHC, FH), f32),
        pltpu.VMEM((CHC, FH), f32),
        pltpu.VMEM_SHARED((NPAD, FH), f32),
        pltpu.SemaphoreType.DMA,
        pltpu.SemaphoreType.DMA,
        pltpu.SemaphoreType.DMA,
        pltpu.SemaphoreType.DMA,
        pltpu.SemaphoreType.DMA,
        pltpu.SemaphoreType.DMA,
    ],
)


def _dense_body(a0_ref, a1_ref, dis_ref, w1_ref, b1_ref, w2_ref, ys_ref):
    d = dis_ref[...]
    a0 = a0_ref[...] * d
    a1 = a1_ref[...] * d
    w1 = w1_ref[...]
    h = (
        jnp.dot(a0, w1[:FH], preferred_element_type=f32,
                precision=lax.Precision.DEFAULT)
        + jnp.dot(a1, w1[FH:], preferred_element_type=f32,
                  precision=lax.Precision.DEFAULT)
    )
    h = jnp.maximum(h + b1_ref[...], 0.0)
    y = jnp.dot(h, w2_ref[...], preferred_element_type=f32,
                precision=lax.Precision.DEFAULT)
    ys_ref[...] = y * d[:, :CPAD]


_dense = pl.pallas_call(
    _dense_body,
    grid=(NPAD // MB,),
    in_specs=[
        pl.BlockSpec((MB, FH), lambda m: (m, 0)),
        pl.BlockSpec((MB, FH), lambda m: (m + NPAD // MB, 0)),
        pl.BlockSpec((MB, FH), lambda m: (m, 0)),
        pl.BlockSpec((NFEAT, HIDDEN), lambda m: (0, 0)),
        pl.BlockSpec((1, HIDDEN), lambda m: (0, 0)),
        pl.BlockSpec((HIDDEN, CPAD), lambda m: (0, 0)),
    ],
    out_specs=pl.BlockSpec((MB, CPAD), lambda m: (m, 0)),
    out_shape=jax.ShapeDtypeStruct((NPAD, CPAD), f32),
)


def _agg2_body(ys_hbm, src_hbm, dst_hbm, zero_hbm, out_hbm,
               src_v, dst_v, r0, r1, r2, r3, acc_sh, s0, s1, s2, s3):
    c = lax.axis_index("c")
    s = lax.axis_index("s")
    wid = c * 16 + s
    pltpu.sync_copy(src_hbm.at[wid], src_v)
    pltpu.sync_copy(dst_hbm.at[wid], dst_v)
    pltpu.sync_copy(zero_hbm, r0)
    base = s * RPT
    for k in range(RPT // CHUNK):
        pltpu.sync_copy(r0, acc_sh.at[pl.ds(base + k * CHUNK, CHUNK)])
    plsc.subcore_barrier()
    _piped_edge_loop(ys_hbm, src_v, dst_v, acc_sh,
                     (r0, r1, r2, r3), (s0, s1, s2, s3), CH_E)
    plsc.subcore_barrier()
    _drain_acc(acc_sh, out_hbm, base, c * NPAD + base, r0)


_agg2_kernel = pl.kernel(
    _agg2_body,
    out_type=jax.ShapeDtypeStruct((2 * NPAD, CPAD), f32),
    mesh=_mesh,
    compiler_params=pltpu.CompilerParams(use_tc_tiling_on_sc=False),
    scratch_types=[
        pltpu.VMEM((CH_E, CHUNK), i32),
        pltpu.VMEM((CH_E, CHUNK), i32),
        pltpu.VMEM((CHUNK, CPAD), f32),
        pltpu.VMEM((CHUNK, CPAD), f32),
        pltpu.VMEM((CHUNK, CPAD), f32),
        pltpu.VMEM((CHUNK, CPAD), f32),
        pltpu.VMEM_SHARED((NPAD, CPAD), f32),
        pltpu.SemaphoreType.DMA,
        pltpu.SemaphoreType.DMA,
        pltpu.SemaphoreType.DMA,
        pltpu.SemaphoreType.DMA,
    ],
)


def _out_body(a0_ref, a1_ref, ys_ref, dis_ref, b2_ref, o_ref):
    z = (a0_ref[...] + a1_ref[...] + ys_ref[...]) * dis_ref[...][:, :CPAD]
    z = z + b2_ref[...]
    col = lax.broadcasted_iota(i32, z.shape, 1)
    z = jnp.where(col < NCLASS, z, -1e30)
    m = jnp.max(z, axis=1, keepdims=True)
    e = jnp.exp(z - m)
    ssum = jnp.sum(e, axis=1, keepdims=True)
    o_ref[...] = z - m - jnp.log(ssum)


_outk = pl.pallas_call(
    _out_body,
    grid=(NPAD // MB,),
    in_specs=[
        pl.BlockSpec((MB, CPAD), lambda m: (m, 0)),
        pl.BlockSpec((MB, CPAD), lambda m: (m + NPAD // MB, 0)),
        pl.BlockSpec((MB, CPAD), lambda m: (m, 0)),
        pl.BlockSpec((MB, FH), lambda m: (m, 0)),
        pl.BlockSpec((1, CPAD), lambda m: (0, 0)),
    ],
    out_specs=pl.BlockSpec((MB, CPAD), lambda m: (m, 0)),
    out_shape=jax.ShapeDtypeStruct((NPAD, CPAD), f32),
)


def kernel(x, edge_index, W1, b1, W2, b2):
    src = edge_index[0].astype(i32)
    dst = edge_index[1].astype(i32)
    n_pad_edges = EPAD - N_EDGES
    pad_idx = jnp.arange(n_pad_edges, dtype=i32)
    src_pad = jnp.concatenate([src, pad_idx % N_NODES])
    dst_pad = jnp.concatenate([dst, DUMMY + (pad_idx % 128)])
    srcC = src_pad.reshape(16, NGC, GCC, CHC)
    dstC = dst_pad.reshape(16, NGC, GCC, CHC)
    srcE = src_pad.reshape(32, CH_E, CHUNK)
    dstE = dst_pad.reshape(32, CH_E, CHUNK)
    dstA = dst_pad.reshape(32, CH_E * CHUNK)
    x_pad = jnp.concatenate([x, jnp.zeros((NPAD - N_NODES, NFEAT), f32)])
    w2p = jnp.pad(W2, ((0, 0), (0, CPAD - NCLASS)))
    b2p = jnp.pad(b2, (0, CPAD - NCLASS)).reshape(1, CPAD)
    b1r = b1.reshape(1, HIDDEN)
    zero64 = jnp.zeros((CHUNK, CPAD), f32)

    degp = _deg_kernel(dstA)
    xs0, xs1, dis2d = _prep(degp, x_pad)
    agg = _agg1_kernel(xs0, xs1, srcC, dstC)
    ys = _dense(agg, agg, dis2d, W1, b1r, w2p)
    acc2 = _agg2_kernel(ys, srcE, dstE, zero64)
    o = _outk(acc2, acc2, ys, dis2d, b2p)
    return o[:N_NODES, :NCLASS]
